# Initial kernel scaffold; baseline (speedup 1.0000x reference)
import jax
import jax.numpy as jnp
from jax.experimental import pallas as pl


def kernel(s, v, radial_embeddings_1, radial_embeddings_2, f_cut_1, f_cut_2, unit_vectors_1, unit_vectors_2, edge_index, W1, b1, W2, b2, Wr, br):
    raise NotImplementedError("write your pallas kernel here")


# speedup vs baseline: 6.0276x; 6.0276x over previous
import functools

import jax
import jax.numpy as jnp
from jax import lax
from jax.experimental import pallas as pl
from jax.experimental.pallas import tpu as pltpu
from jax.experimental.pallas import tpu_sc as plsc

N = 10000
E = 320000
F = 128
R = 32
P = 4
FQ = F // P
GW = 6 * FQ
VW = 3 * FQ
GP = 256
VP = 128
NB = 1000
EB = 2000
NCORES = 2
NSUB = 16
NWORK = NCORES * NSUB
EPW = E // NWORK
C = 40
NCH = EPW // C
NP = 10240
RPT = NP // NSUB


def _phi_v_body(s_ref, v_ref, w1_ref, b1_ref, w2_ref, b2_ref, phi_ref, v4_ref):
    h = jax.nn.silu(s_ref[...] @ w1_ref[...] + b1_ref[...])
    phi = h @ w2_ref[...] + b2_ref[...]
    vblk = v_ref[...]
    pad_g = jnp.zeros((phi.shape[0], GP - GW), jnp.float32)
    pad_v = jnp.zeros((phi.shape[0], VP - VW), jnp.float32)
    for p in range(P):
        phi_ref[p] = jnp.concatenate(
            [phi[:, p * GW:(p + 1) * GW], pad_g], axis=1)
        v4_ref[p] = jnp.concatenate(
            [vblk[:, d * F + p * FQ: d * F + (p + 1) * FQ] for d in range(3)]
            + [pad_v], axis=1)


def _wfilt_body(r1_ref, r2_ref, f1_ref, f2_ref, wr_ref, br_ref, wf_ref):
    wr = wr_ref[...]
    br = br_ref[...]
    wf = ((r1_ref[...] @ wr + br) * f1_ref[...]
          + (r2_ref[...] @ wr + br) * f2_ref[...])
    for p in range(P):
        wf_ref[p] = wf[:, p * GW:(p + 1) * GW]


def _combine_body(s_ref, v_ref, p0_ref, p1_ref, p2_ref, p3_ref,
                  os_ref, ov_ref):
    parts = [p0_ref, p1_ref, p2_ref, p3_ref]
    ps = [pr[0] + pr[1] for pr in parts]
    os_ref[...] = s_ref[...] + jnp.concatenate(
        [ps[p][:, 0:FQ] for p in range(P)], axis=1)
    ov_ref[...] = v_ref[...] + jnp.concatenate(
        [ps[p][:, FQ * (d + 1): FQ * (d + 2)] for d in range(3)
         for p in range(P)], axis=1)


def _scatter_add_rows(src, acc, idx):
    pltpu.sync_copy(src, acc.at[idx], add=True)


def _sc_body(*refs):
    (phi_refs, v_refs, wf_refs) = (refs[0:P], refs[P:2 * P], refs[2 * P:3 * P])
    (idxi_hbm, idxj_hbm, uv1_hbm, uv2_hbm, zeros_hbm) = refs[3 * P:3 * P + 5]
    out_refs = refs[3 * P + 5:4 * P + 5]
    (idxj, idxi, phib, wfb, vb, uv1b, uv2b, outb, acc,
     sem_g1, sem_g2) = refs[4 * P + 5:]

    cid = lax.axis_index("c")
    sid = lax.axis_index("s")
    wid = sid * NCORES + cid
    row0 = sid * RPT

    def edge(e, carry):
        u1w = uv1b[pl.ds(3 * e, 16)]
        u2w = uv2b[pl.ds(3 * e, 16)]

        def bcast(win, d):
            dn = lax.GatherDimensionNumbers(
                offset_dims=(), collapsed_slice_dims=(0,),
                start_index_map=(0,))
            return lax.gather(
                win, jnp.full((16, 1), d, jnp.int32), dn, slice_sizes=(1,),
                mode=lax.GatherScatterMode.PROMISE_IN_BOUNDS)
        u1 = [bcast(u1w, d) for d in range(3)]
        u2 = [bcast(u2w, d) for d in range(3)]
        x = {}
        for g in range(6):
            for k in range(2):
                sl = pl.ds(g * FQ + k * 16, 16)
                x[(g, k)] = phib[e, sl] * wfb[e, sl]
        for k in range(2):
            outb[e, pl.ds(k * 16, 16)] = x[(0, k)]
        vj = {}
        for d in range(3):
            for k in range(2):
                vj[(d, k)] = vb[e, pl.ds(d * FQ + k * 16, 16)]
        for d in range(3):
            a = (d + 1) % 3
            b = (d + 2) % 3
            for k in range(2):
                c1 = vj[(a, k)] * u1[b] - vj[(b, k)] * u1[a]
                c2 = vj[(a, k)] * u2[b] - vj[(b, k)] * u2[a]
                xv = (vj[(d, k)] * x[(1, k)] + x[(2, k)] * u1[d]
                      + x[(3, k)] * u2[d] + x[(4, k)] * c1 + x[(5, k)] * c2)
                outb[e, pl.ds(FQ + d * FQ + k * 16, 16)] = xv
        return carry

    for p in range(P):
        pltpu.sync_copy(zeros_hbm.at[pl.ds(row0, RPT)],
                        acc.at[pl.ds(row0, RPT)])
        plsc.subcore_barrier()

        def chunk(c, carry):
            e0 = wid * EPW + c * C
            pltpu.sync_copy(idxj_hbm.at[pl.ds(e0, C)], idxj)
            pltpu.sync_copy(idxi_hbm.at[pl.ds(e0, C)], idxi)
            g1 = pltpu.async_copy(phi_refs[p].at[idxj], phib, sem_g1)
            g2 = pltpu.async_copy(v_refs[p].at[idxj], vb, sem_g2)
            pltpu.sync_copy(wf_refs[p].at[pl.ds(e0, C)], wfb)
            pltpu.sync_copy(uv1_hbm.at[pl.ds(3 * e0, 3 * C)],
                            uv1b.at[pl.ds(0, 3 * C)])
            pltpu.sync_copy(uv2_hbm.at[pl.ds(3 * e0, 3 * C)],
                            uv2b.at[pl.ds(0, 3 * C)])
            g1.wait()
            g2.wait()
            lax.fori_loop(0, C, edge, 0)
            _scatter_add_rows(outb, acc, idxi)
            return carry

        lax.fori_loop(0, NCH, chunk, 0)
        plsc.subcore_barrier()
        pltpu.sync_copy(acc.at[pl.ds(row0, RPT)],
                        out_refs[p].at[cid, pl.ds(row0, RPT)])


def _make_sc_kernel():
    mesh = plsc.VectorSubcoreMesh(core_axis_name="c", subcore_axis_name="s",
                                  num_cores=NCORES, num_subcores=NSUB)
    out_type = [jax.ShapeDtypeStruct((NCORES, NP, F), jnp.float32)
                for _ in range(P)]
    scratch = [
        pltpu.VMEM((C,), jnp.int32),
        pltpu.VMEM((C,), jnp.int32),
        pltpu.VMEM((C, GP), jnp.float32),
        pltpu.VMEM((C, GW), jnp.float32),
        pltpu.VMEM((C, VP), jnp.float32),
        pltpu.VMEM((3 * C + 16,), jnp.float32),
        pltpu.VMEM((3 * C + 16,), jnp.float32),
        pltpu.VMEM((C, F), jnp.float32),
        pltpu.VMEM_SHARED((NP, F), jnp.float32),
        pltpu.SemaphoreType.DMA,
        pltpu.SemaphoreType.DMA,
    ]
    return pl.kernel(_sc_body, out_type=out_type, mesh=mesh,
                     scratch_types=scratch)


def kernel(s, v, radial_embeddings_1, radial_embeddings_2, f_cut_1, f_cut_2,
           unit_vectors_1, unit_vectors_2, edge_index, W1, b1, W2, b2, Wr,
           br):
    s2 = s.reshape(N, F)
    v2 = v.reshape(N, 3 * F)
    r1 = radial_embeddings_1.reshape(E, R)
    r2 = radial_embeddings_2.reshape(E, R)
    f1 = f_cut_1.reshape(E, 1)
    f2 = f_cut_2.reshape(E, 1)
    uv1 = unit_vectors_1.reshape(3 * E)
    uv2 = unit_vectors_2.reshape(3 * E)
    ei = edge_index.astype(jnp.int32)
    idx_i = ei[0]
    idx_j = ei[1]
    perm = (jnp.arange(P)[:, None, None] * FQ
            + jnp.arange(6)[None, :, None] * F
            + jnp.arange(FQ)[None, None, :]).reshape(-1)
    W2p = W2[:, perm]
    b2p = b2[perm].reshape(1, 6 * F)
    Wrp = Wr[:, perm]
    brp = br[perm].reshape(1, 6 * F)
    b1r = b1.reshape(1, F)
    zeros = jnp.zeros((NP, F), jnp.float32)

    phi4, v4 = pl.pallas_call(
        _phi_v_body,
        grid=(N // NB,),
        in_specs=[
            pl.BlockSpec((NB, F), lambda i: (i, 0)),
            pl.BlockSpec((NB, 3 * F), lambda i: (i, 0)),
            pl.BlockSpec((F, F), lambda i: (0, 0)),
            pl.BlockSpec((1, F), lambda i: (0, 0)),
            pl.BlockSpec((F, 6 * F), lambda i: (0, 0)),
            pl.BlockSpec((1, 6 * F), lambda i: (0, 0)),
        ],
        out_specs=[
            pl.BlockSpec((P, NB, GP), lambda i: (0, i, 0)),
            pl.BlockSpec((P, NB, VP), lambda i: (0, i, 0)),
        ],
        out_shape=[
            jax.ShapeDtypeStruct((P, N, GP), jnp.float32),
            jax.ShapeDtypeStruct((P, N, VP), jnp.float32),
        ],
    )(s2, v2, W1, b1r, W2p, b2p)

    wf4 = pl.pallas_call(
        _wfilt_body,
        grid=(E // EB,),
        in_specs=[
            pl.BlockSpec((EB, R), lambda i: (i, 0)),
            pl.BlockSpec((EB, R), lambda i: (i, 0)),
            pl.BlockSpec((EB, 1), lambda i: (i, 0)),
            pl.BlockSpec((EB, 1), lambda i: (i, 0)),
            pl.BlockSpec((R, 6 * F), lambda i: (0, 0)),
            pl.BlockSpec((1, 6 * F), lambda i: (0, 0)),
        ],
        out_specs=[pl.BlockSpec((P, EB, GW), lambda i: (0, i, 0))],
        out_shape=[jax.ShapeDtypeStruct((P, E, GW), jnp.float32)],
    )(r1, r2, f1, f2, Wrp, brp)
    wf4 = wf4[0]

    sc = _make_sc_kernel()
    parts = sc(phi4[0], phi4[1], phi4[2], phi4[3],
               v4[0], v4[1], v4[2], v4[3],
               wf4[0], wf4[1], wf4[2], wf4[3],
               idx_i, idx_j, uv1, uv2, zeros)

    out_s, out_v = pl.pallas_call(
        _combine_body,
        grid=(N // NB,),
        in_specs=[
            pl.BlockSpec((NB, F), lambda i: (i, 0)),
            pl.BlockSpec((NB, 3 * F), lambda i: (i, 0)),
        ] + [pl.BlockSpec((NCORES, NB, F), lambda i: (0, i, 0))] * P,
        out_specs=[
            pl.BlockSpec((NB, F), lambda i: (i, 0)),
            pl.BlockSpec((NB, 3 * F), lambda i: (i, 0)),
        ],
        out_shape=[
            jax.ShapeDtypeStruct((N, F), jnp.float32),
            jax.ShapeDtypeStruct((N, 3 * F), jnp.float32),
        ],
    )(s2, v2, parts[0], parts[1], parts[2], parts[3])

    return (out_s.reshape(N, 1, F), out_v.reshape(N, 3, F))


# speedup vs baseline: 7.1214x; 1.1815x over previous
import functools

import jax
import jax.numpy as jnp
from jax import lax
from jax.experimental import pallas as pl
from jax.experimental.pallas import tpu as pltpu
from jax.experimental.pallas import tpu_sc as plsc

N = 10000
E = 320000
F = 128
R = 32
P = 4
FQ = F // P
GW = 6 * FQ
VW = 3 * FQ
GP = 256
VP = 128
NB = 1000
EB = 2000
NCORES = 2
NSUB = 16
NWORK = NCORES * NSUB
EPW = E // NWORK
C = 40
NCH = EPW // C
NP = 10240
RPT = NP // NSUB


def _phi_v_body(s_ref, v_ref, w1_ref, b1_ref, w2_ref, b2_ref, phi_ref, v4_ref):
    h = jax.nn.silu(s_ref[...] @ w1_ref[...] + b1_ref[...])
    phi = h @ w2_ref[...] + b2_ref[...]
    vblk = v_ref[...]
    pad_g = jnp.zeros((phi.shape[0], GP - GW), jnp.float32)
    pad_v = jnp.zeros((phi.shape[0], VP - VW), jnp.float32)
    for p in range(P):
        phi_ref[p] = jnp.concatenate(
            [phi[:, p * GW:(p + 1) * GW], pad_g], axis=1)
        v4_ref[p] = jnp.concatenate(
            [vblk[:, d * F + p * FQ: d * F + (p + 1) * FQ] for d in range(3)]
            + [pad_v], axis=1)


def _wfilt_body(r1_ref, r2_ref, f1_ref, f2_ref, wr_ref, br_ref, wf_ref):
    wr = wr_ref[...]
    br = br_ref[...]
    wf = ((r1_ref[...] @ wr + br) * f1_ref[...]
          + (r2_ref[...] @ wr + br) * f2_ref[...])
    for p in range(P):
        wf_ref[p] = wf[:, p * GW:(p + 1) * GW]


def _combine_body(s_ref, v_ref, p0_ref, p1_ref, p2_ref, p3_ref,
                  os_ref, ov_ref):
    parts = [p0_ref, p1_ref, p2_ref, p3_ref]
    ps = [pr[0] + pr[1] for pr in parts]
    os_ref[...] = s_ref[...] + jnp.concatenate(
        [ps[p][:, 0:FQ] for p in range(P)], axis=1)
    ov_ref[...] = v_ref[...] + jnp.concatenate(
        [ps[p][:, FQ * (d + 1): FQ * (d + 2)] for d in range(3)
         for p in range(P)], axis=1)


def _scatter_add_rows(src, acc, idx):
    pltpu.sync_copy(src, acc.at[idx], add=True)


def _sc_body(*refs):
    (phi_refs, v_refs, wf_refs) = (refs[0:P], refs[P:2 * P], refs[2 * P:3 * P])
    (idxi_hbm, idxj_hbm, uv1_hbm, uv2_hbm, zeros_hbm) = refs[3 * P:3 * P + 5]
    out_refs = refs[3 * P + 5:4 * P + 5]
    (idxj, idxi, phib, wfb, vb, uv1b, uv2b, outb, acc,
     sem_g1, sem_g2) = refs[4 * P + 5:]

    cid = lax.axis_index("c")
    sid = lax.axis_index("s")
    wid = sid * NCORES + cid
    row0 = sid * RPT

    def edge(e, carry):
        u1w = uv1b[pl.ds(3 * e, 16)]
        u2w = uv2b[pl.ds(3 * e, 16)]

        def bcast(win, d):
            dn = lax.GatherDimensionNumbers(
                offset_dims=(), collapsed_slice_dims=(0,),
                start_index_map=(0,))
            return lax.gather(
                win, jnp.full((16, 1), d, jnp.int32), dn, slice_sizes=(1,),
                mode=lax.GatherScatterMode.PROMISE_IN_BOUNDS)
        u1 = [bcast(u1w, d) for d in range(3)]
        u2 = [bcast(u2w, d) for d in range(3)]
        x = {}
        for g in range(6):
            for k in range(2):
                sl = pl.ds(g * FQ + k * 16, 16)
                x[(g, k)] = phib[e, sl] * wfb[e, sl]
        for k in range(2):
            outb[e, pl.ds(k * 16, 16)] = x[(0, k)]
        vj = {}
        for d in range(3):
            for k in range(2):
                vj[(d, k)] = vb[e, pl.ds(d * FQ + k * 16, 16)]
        t = {}
        for d in range(3):
            for k in range(2):
                t[(d, k)] = x[(4, k)] * u1[d] + x[(5, k)] * u2[d]
        for d in range(3):
            a = (d + 1) % 3
            b = (d + 2) % 3
            for k in range(2):
                xv = (vj[(d, k)] * x[(1, k)] + x[(2, k)] * u1[d]
                      + x[(3, k)] * u2[d]
                      + vj[(a, k)] * t[(b, k)] - vj[(b, k)] * t[(a, k)])
                outb[e, pl.ds(FQ + d * FQ + k * 16, 16)] = xv
        return carry

    for p in range(P):
        pltpu.sync_copy(zeros_hbm.at[pl.ds(row0, RPT)],
                        acc.at[pl.ds(row0, RPT)])
        plsc.subcore_barrier()

        def chunk(c, carry):
            e0 = wid * EPW + c * C
            loads = [
                pltpu.async_copy(idxj_hbm.at[pl.ds(e0, C)], idxj, sem_g1),
                pltpu.async_copy(idxi_hbm.at[pl.ds(e0, C)], idxi, sem_g1),
                pltpu.async_copy(wf_refs[p].at[pl.ds(e0, C)], wfb, sem_g1),
                pltpu.async_copy(uv1_hbm.at[pl.ds(3 * e0, 3 * C)],
                                 uv1b.at[pl.ds(0, 3 * C)], sem_g1),
                pltpu.async_copy(uv2_hbm.at[pl.ds(3 * e0, 3 * C)],
                                 uv2b.at[pl.ds(0, 3 * C)], sem_g1),
            ]
            for ld in loads:
                ld.wait()
            g1 = pltpu.async_copy(phi_refs[p].at[idxj], phib, sem_g2)
            g2 = pltpu.async_copy(v_refs[p].at[idxj], vb, sem_g2)
            g1.wait()
            g2.wait()
            lax.fori_loop(0, C, edge, 0, unroll=2)
            _scatter_add_rows(outb, acc, idxi)
            return carry

        lax.fori_loop(0, NCH, chunk, 0)
        plsc.subcore_barrier()
        pltpu.sync_copy(acc.at[pl.ds(row0, RPT)],
                        out_refs[p].at[cid, pl.ds(row0, RPT)])


def _make_sc_kernel():
    mesh = plsc.VectorSubcoreMesh(core_axis_name="c", subcore_axis_name="s",
                                  num_cores=NCORES, num_subcores=NSUB)
    out_type = [jax.ShapeDtypeStruct((NCORES, NP, F), jnp.float32)
                for _ in range(P)]
    scratch = [
        pltpu.VMEM((C,), jnp.int32),
        pltpu.VMEM((C,), jnp.int32),
        pltpu.VMEM((C, GP), jnp.float32),
        pltpu.VMEM((C, GW), jnp.float32),
        pltpu.VMEM((C, VP), jnp.float32),
        pltpu.VMEM((3 * C + 16,), jnp.float32),
        pltpu.VMEM((3 * C + 16,), jnp.float32),
        pltpu.VMEM((C, F), jnp.float32),
        pltpu.VMEM_SHARED((NP, F), jnp.float32),
        pltpu.SemaphoreType.DMA,
        pltpu.SemaphoreType.DMA,
    ]
    return pl.kernel(_sc_body, out_type=out_type, mesh=mesh,
                     scratch_types=scratch)


def kernel(s, v, radial_embeddings_1, radial_embeddings_2, f_cut_1, f_cut_2,
           unit_vectors_1, unit_vectors_2, edge_index, W1, b1, W2, b2, Wr,
           br):
    s2 = s.reshape(N, F)
    v2 = v.reshape(N, 3 * F)
    r1 = radial_embeddings_1.reshape(E, R)
    r2 = radial_embeddings_2.reshape(E, R)
    f1 = f_cut_1.reshape(E, 1)
    f2 = f_cut_2.reshape(E, 1)
    uv1 = unit_vectors_1.reshape(3 * E)
    uv2 = unit_vectors_2.reshape(3 * E)
    ei = edge_index.astype(jnp.int32)
    idx_i = ei[0]
    idx_j = ei[1]
    perm = (jnp.arange(P)[:, None, None] * FQ
            + jnp.arange(6)[None, :, None] * F
            + jnp.arange(FQ)[None, None, :]).reshape(-1)
    W2p = W2[:, perm]
    b2p = b2[perm].reshape(1, 6 * F)
    Wrp = Wr[:, perm]
    brp = br[perm].reshape(1, 6 * F)
    b1r = b1.reshape(1, F)
    zeros = jnp.zeros((NP, F), jnp.float32)

    phi4, v4 = pl.pallas_call(
        _phi_v_body,
        grid=(N // NB,),
        in_specs=[
            pl.BlockSpec((NB, F), lambda i: (i, 0)),
            pl.BlockSpec((NB, 3 * F), lambda i: (i, 0)),
            pl.BlockSpec((F, F), lambda i: (0, 0)),
            pl.BlockSpec((1, F), lambda i: (0, 0)),
            pl.BlockSpec((F, 6 * F), lambda i: (0, 0)),
            pl.BlockSpec((1, 6 * F), lambda i: (0, 0)),
        ],
        out_specs=[
            pl.BlockSpec((P, NB, GP), lambda i: (0, i, 0)),
            pl.BlockSpec((P, NB, VP), lambda i: (0, i, 0)),
        ],
        out_shape=[
            jax.ShapeDtypeStruct((P, N, GP), jnp.float32),
            jax.ShapeDtypeStruct((P, N, VP), jnp.float32),
        ],
    )(s2, v2, W1, b1r, W2p, b2p)

    wf4 = pl.pallas_call(
        _wfilt_body,
        grid=(E // EB,),
        in_specs=[
            pl.BlockSpec((EB, R), lambda i: (i, 0)),
            pl.BlockSpec((EB, R), lambda i: (i, 0)),
            pl.BlockSpec((EB, 1), lambda i: (i, 0)),
            pl.BlockSpec((EB, 1), lambda i: (i, 0)),
            pl.BlockSpec((R, 6 * F), lambda i: (0, 0)),
            pl.BlockSpec((1, 6 * F), lambda i: (0, 0)),
        ],
        out_specs=[pl.BlockSpec((P, EB, GW), lambda i: (0, i, 0))],
        out_shape=[jax.ShapeDtypeStruct((P, E, GW), jnp.float32)],
    )(r1, r2, f1, f2, Wrp, brp)
    wf4 = wf4[0]

    sc = _make_sc_kernel()
    parts = sc(phi4[0], phi4[1], phi4[2], phi4[3],
               v4[0], v4[1], v4[2], v4[3],
               wf4[0], wf4[1], wf4[2], wf4[3],
               idx_i, idx_j, uv1, uv2, zeros)

    out_s, out_v = pl.pallas_call(
        _combine_body,
        grid=(N // NB,),
        in_specs=[
            pl.BlockSpec((NB, F), lambda i: (i, 0)),
            pl.BlockSpec((NB, 3 * F), lambda i: (i, 0)),
        ] + [pl.BlockSpec((NCORES, NB, F), lambda i: (0, i, 0))] * P,
        out_specs=[
            pl.BlockSpec((NB, F), lambda i: (i, 0)),
            pl.BlockSpec((NB, 3 * F), lambda i: (i, 0)),
        ],
        out_shape=[
            jax.ShapeDtypeStruct((N, F), jnp.float32),
            jax.ShapeDtypeStruct((N, 3 * F), jnp.float32),
        ],
    )(s2, v2, parts[0], parts[1], parts[2], parts[3])

    return (out_s.reshape(N, 1, F), out_v.reshape(N, 3, F))


# speedup vs baseline: 11.4133x; 1.6027x over previous
import functools

import jax
import jax.numpy as jnp
import numpy as np
from jax import lax
from jax.experimental import pallas as pl
from jax.experimental.pallas import tpu as pltpu
from jax.experimental.pallas import tpu_sc as plsc

N = 10000
E = 320000
F = 128
R = 32
P = 4
FQ = F // P
GW = 6 * FQ
VW = 3 * FQ
GP = 128
WP = 96
VP = 128
NB = 1000
EB = 2000
NCORES = 2
NSUB = 16
NWORK = NCORES * NSUB
EPW = E // NWORK
C = 40
NCH = EPW // C
NP = 10240
RPT = NP // NSUB

_MASK_HI = -65536


def _pack16(lo, hi):
    lob = lax.bitcast_convert_type(lo, jnp.int32)
    hib = lax.bitcast_convert_type(hi, jnp.int32)
    return jnp.bitwise_or(
        lax.shift_right_logical(lob + 32768, 16),
        jnp.bitwise_and(hib + 32768, _MASK_HI))


def _phi_v_body(s_ref, v_ref, w1_ref, b1_ref, w2_ref, b2_ref, phi_ref,
                v4_ref):
    h = jax.nn.silu(s_ref[...] @ w1_ref[...] + b1_ref[...])
    phi = h @ w2_ref[...] + b2_ref[...]
    vblk = v_ref[...]
    pad_g = jnp.zeros((phi.shape[0], GP - GW // 2), jnp.int32)
    pad_v = jnp.zeros((phi.shape[0], VP - VW), jnp.float32)
    for p in range(P):
        blk = phi[:, p * GW:(p + 1) * GW]
        phi_ref[p] = jnp.concatenate(
            [_pack16(blk[:, :GW // 2], blk[:, GW // 2:]), pad_g], axis=1)
        v4_ref[p] = jnp.concatenate(
            [vblk[:, d * F + p * FQ: d * F + (p + 1) * FQ] for d in range(3)]
            + [pad_v], axis=1)


def _wfilt_body(r1_ref, r2_ref, f1_ref, f2_ref, wr_ref, br_ref, wf_ref):
    wr = wr_ref[...]
    br = br_ref[...]
    wf = ((r1_ref[...] @ wr + br) * f1_ref[...]
          + (r2_ref[...] @ wr + br) * f2_ref[...])
    for p in range(P):
        blk = wf[:, p * GW:(p + 1) * GW]
        wf_ref[p] = _pack16(blk[:, :GW // 2], blk[:, GW // 2:])


def _combine_body(s_ref, v_ref, p0_ref, p1_ref, p2_ref, p3_ref,
                  os_ref, ov_ref):
    parts = [p0_ref, p1_ref, p2_ref, p3_ref]
    ps = [pr[0] + pr[1] for pr in parts]
    os_ref[...] = s_ref[...] + jnp.concatenate(
        [ps[p][:, 0:FQ] for p in range(P)], axis=1)
    ov_ref[...] = v_ref[...] + jnp.concatenate(
        [ps[p][:, FQ * (d + 1): FQ * (d + 2)] for d in range(3)
         for p in range(P)], axis=1)


def _scatter_add_rows(src, acc, idx):
    pltpu.sync_copy(src, acc.at[idx], add=True)


def _halves(buf, e, g):
    r = buf[e, pl.ds(g * 16, 16)]
    lo = plsc.bitcast(lax.shift_left(r, jnp.full((16,), 16, jnp.int32)),
                      jnp.float32)
    hi = plsc.bitcast(lax.bitwise_and(r, jnp.full((16,), _MASK_HI,
                                                  jnp.int32)),
                      jnp.float32)
    return lo, hi


def _sc_body(*refs):
    (phi_refs, v_refs, wf_refs) = (refs[0:P], refs[P:2 * P], refs[2 * P:3 * P])
    (idxi_hbm, idxj_hbm, uv1_hbm, uv2_hbm, zeros_hbm) = refs[3 * P:3 * P + 5]
    out_refs = refs[3 * P + 5:4 * P + 5]
    sc = refs[4 * P + 5:]
    idxj = sc[0:2]
    idxi = sc[2:4]
    phib = sc[4:6]
    wfb = sc[6:8]
    vb = sc[8:10]
    uv1b = sc[10:12]
    uv2b = sc[12:14]
    outb = sc[14]
    acc = sc[15]
    sem_l = sc[16:18]
    sem_g = sc[18:20]

    cid = lax.axis_index("c")
    sid = lax.axis_index("s")
    wid = sid * NCORES + cid
    row0 = sid * RPT

    def make_edge(phib_, wfb_, vb_, uv1b_, uv2b_):
        def edge(e, carry):
            u1w = uv1b_[pl.ds(3 * e, 16)]
            u2w = uv2b_[pl.ds(3 * e, 16)]

            def bcast(win, d):
                dn = lax.GatherDimensionNumbers(
                    offset_dims=(), collapsed_slice_dims=(0,),
                    start_index_map=(0,))
                return lax.gather(
                    win, jnp.full((16, 1), d, jnp.int32), dn,
                    slice_sizes=(1,),
                    mode=lax.GatherScatterMode.PROMISE_IN_BOUNDS)
            u1 = [bcast(u1w, d) for d in range(3)]
            u2 = [bcast(u2w, d) for d in range(3)]
            x = {}
            for g in range(6):
                plo, phi_ = _halves(phib_, e, g)
                wlo, whi = _halves(wfb_, e, g)
                x[(g, 0)] = plo * wlo
                x[(g, 1)] = phi_ * whi
            for k in range(2):
                outb[e, pl.ds(k * 16, 16)] = x[(0, k)]
            vj = {}
            for d in range(3):
                for k in range(2):
                    vj[(d, k)] = vb_[e, pl.ds(d * FQ + k * 16, 16)]
            t = {}
            for d in range(3):
                for k in range(2):
                    t[(d, k)] = x[(4, k)] * u1[d] + x[(5, k)] * u2[d]
            for d in range(3):
                a = (d + 1) % 3
                b = (d + 2) % 3
                for k in range(2):
                    xv = (vj[(d, k)] * x[(1, k)] + x[(2, k)] * u1[d]
                          + x[(3, k)] * u2[d]
                          + vj[(a, k)] * t[(b, k)] - vj[(b, k)] * t[(a, k)])
                    outb[e, pl.ds(FQ + d * FQ + k * 16, 16)] = xv
            return carry
        return edge

    for p in range(P):
        pltpu.sync_copy(zeros_hbm.at[pl.ds(row0, RPT)],
                        acc.at[pl.ds(row0, RPT)])
        plsc.subcore_barrier()

        def load_pairs(c, B):
            e0 = wid * EPW + c * C
            return [
                (idxj_hbm.at[pl.ds(e0, C)], idxj[B]),
                (idxi_hbm.at[pl.ds(e0, C)], idxi[B]),
                (wf_refs[p].at[pl.ds(e0, C)], wfb[B]),
                (uv1_hbm.at[pl.ds(3 * e0, 3 * C)], uv1b[B].at[pl.ds(0, 3 * C)]),
                (uv2_hbm.at[pl.ds(3 * e0, 3 * C)], uv2b[B].at[pl.ds(0, 3 * C)]),
            ]

        def issue_loads(c, B):
            for src, dst in load_pairs(c, B):
                pltpu.async_copy(src, dst, sem_l[B])

        def wait_loads(c, B):
            for src, dst in load_pairs(c, B):
                pltpu.make_async_copy(src, dst, sem_l[B]).wait()

        def issue_gathers(B):
            pltpu.async_copy(phi_refs[p].at[idxj[B]], phib[B], sem_g[B])
            pltpu.async_copy(v_refs[p].at[idxj[B]], vb[B], sem_g[B])

        def wait_gathers(B):
            pltpu.make_async_copy(
                phi_refs[p].at[idxj[B]], phib[B], sem_g[B]).wait()
            pltpu.make_async_copy(
                v_refs[p].at[idxj[B]], vb[B], sem_g[B]).wait()

        def compute(B):
            lax.fori_loop(
                0, C,
                make_edge(phib[B], wfb[B], vb[B], uv1b[B], uv2b[B]),
                0, unroll=2)
            _scatter_add_rows(outb, acc, idxi[B])

        issue_loads(0, 0)
        wait_loads(0, 0)
        issue_gathers(0)
        issue_loads(1, 1)

        def pair(i, carry):
            c0 = 2 * i
            wait_loads(c0 + 1, 1)
            issue_gathers(1)
            wait_gathers(0)
            compute(0)
            issue_loads(c0 + 2, 0)
            wait_gathers(1)
            compute(1)
            issue_loads(c0 + 3, 1)
            wait_loads(c0 + 2, 0)
            issue_gathers(0)
            return carry

        lax.fori_loop(0, NCH // 2 - 1, pair, 0)
        wait_loads(NCH - 1, 1)
        issue_gathers(1)
        wait_gathers(0)
        compute(0)
        wait_gathers(1)
        compute(1)

        plsc.subcore_barrier()
        pltpu.sync_copy(acc.at[pl.ds(row0, RPT)],
                        out_refs[p].at[cid, pl.ds(row0, RPT)])


def _make_sc_kernel():
    mesh = plsc.VectorSubcoreMesh(core_axis_name="c", subcore_axis_name="s",
                                  num_cores=NCORES, num_subcores=NSUB)
    out_type = [jax.ShapeDtypeStruct((NCORES, NP, F), jnp.float32)
                for _ in range(P)]
    scratch = (
        [pltpu.VMEM((C,), jnp.int32) for _ in range(2)]
        + [pltpu.VMEM((C,), jnp.int32) for _ in range(2)]
        + [pltpu.VMEM((C, GP), jnp.int32) for _ in range(2)]
        + [pltpu.VMEM((C, WP), jnp.int32) for _ in range(2)]
        + [pltpu.VMEM((C, VP), jnp.float32) for _ in range(2)]
        + [pltpu.VMEM((3 * C + 16,), jnp.float32) for _ in range(2)]
        + [pltpu.VMEM((3 * C + 16,), jnp.float32) for _ in range(2)]
        + [
            pltpu.VMEM((C, F), jnp.float32),
            pltpu.VMEM_SHARED((NP, F), jnp.float32),
            pltpu.SemaphoreType.DMA,
            pltpu.SemaphoreType.DMA,
            pltpu.SemaphoreType.DMA,
            pltpu.SemaphoreType.DMA,
        ])
    return pl.kernel(
        _sc_body, out_type=out_type, mesh=mesh, scratch_types=scratch,
        compiler_params=pltpu.CompilerParams(needs_layout_passes=False))


def _packed_perm():
    perm = (np.arange(6)[None, None, :, None] * F
            + np.arange(P)[:, None, None, None] * FQ
            + np.arange(2)[None, :, None, None] * 16
            + np.arange(16)[None, None, None, :])
    return perm.reshape(-1).astype(np.int32)


def kernel(s, v, radial_embeddings_1, radial_embeddings_2, f_cut_1, f_cut_2,
           unit_vectors_1, unit_vectors_2, edge_index, W1, b1, W2, b2, Wr,
           br):
    s2 = s.reshape(N, F)
    v2 = v.reshape(N, 3 * F)
    r1 = radial_embeddings_1.reshape(E, R)
    r2 = radial_embeddings_2.reshape(E, R)
    f1 = f_cut_1.reshape(E, 1)
    f2 = f_cut_2.reshape(E, 1)
    uv1 = unit_vectors_1.reshape(3 * E)
    uv2 = unit_vectors_2.reshape(3 * E)
    ei = edge_index.astype(jnp.int32)
    idx_i = ei[0]
    idx_j = ei[1]
    perm = _packed_perm()
    W2p = W2[:, perm]
    b2p = b2[perm].reshape(1, 6 * F)
    Wrp = Wr[:, perm]
    brp = br[perm].reshape(1, 6 * F)
    b1r = b1.reshape(1, F)
    zeros = jnp.zeros((NP, F), jnp.float32)

    phi4, v4 = pl.pallas_call(
        _phi_v_body,
        grid=(N // NB,),
        in_specs=[
            pl.BlockSpec((NB, F), lambda i: (i, 0)),
            pl.BlockSpec((NB, 3 * F), lambda i: (i, 0)),
            pl.BlockSpec((F, F), lambda i: (0, 0)),
            pl.BlockSpec((1, F), lambda i: (0, 0)),
            pl.BlockSpec((F, 6 * F), lambda i: (0, 0)),
            pl.BlockSpec((1, 6 * F), lambda i: (0, 0)),
        ],
        out_specs=[
            pl.BlockSpec((P, NB, GP), lambda i: (0, i, 0)),
            pl.BlockSpec((P, NB, VP), lambda i: (0, i, 0)),
        ],
        out_shape=[
            jax.ShapeDtypeStruct((P, N, GP), jnp.int32),
            jax.ShapeDtypeStruct((P, N, VP), jnp.float32),
        ],
    )(s2, v2, W1, b1r, W2p, b2p)

    wf4 = pl.pallas_call(
        _wfilt_body,
        grid=(E // EB,),
        in_specs=[
            pl.BlockSpec((EB, R), lambda i: (i, 0)),
            pl.BlockSpec((EB, R), lambda i: (i, 0)),
            pl.BlockSpec((EB, 1), lambda i: (i, 0)),
            pl.BlockSpec((EB, 1), lambda i: (i, 0)),
            pl.BlockSpec((R, 6 * F), lambda i: (0, 0)),
            pl.BlockSpec((1, 6 * F), lambda i: (0, 0)),
        ],
        out_specs=[pl.BlockSpec((P, EB, WP), lambda i: (0, i, 0))],
        out_shape=[jax.ShapeDtypeStruct((P, E, WP), jnp.int32)],
    )(r1, r2, f1, f2, Wrp, brp)
    wf4 = wf4[0]

    sc = _make_sc_kernel()
    parts = sc(phi4[0], phi4[1], phi4[2], phi4[3],
               v4[0], v4[1], v4[2], v4[3],
               wf4[0], wf4[1], wf4[2], wf4[3],
               idx_i, idx_j, uv1, uv2, zeros)

    out_s, out_v = pl.pallas_call(
        _combine_body,
        grid=(N // NB,),
        in_specs=[
            pl.BlockSpec((NB, F), lambda i: (i, 0)),
            pl.BlockSpec((NB, 3 * F), lambda i: (i, 0)),
        ] + [pl.BlockSpec((NCORES, NB, F), lambda i: (0, i, 0))] * P,
        out_specs=[
            pl.BlockSpec((NB, F), lambda i: (i, 0)),
            pl.BlockSpec((NB, 3 * F), lambda i: (i, 0)),
        ],
        out_shape=[
            jax.ShapeDtypeStruct((N, F), jnp.float32),
            jax.ShapeDtypeStruct((N, 3 * F), jnp.float32),
        ],
    )(s2, v2, parts[0], parts[1], parts[2], parts[3])

    return (out_s.reshape(N, 1, F), out_v.reshape(N, 3, F))


# speedup vs baseline: 11.4724x; 1.0052x over previous
import functools

import jax
import jax.numpy as jnp
import numpy as np
from jax import lax
from jax.experimental import pallas as pl
from jax.experimental.pallas import tpu as pltpu
from jax.experimental.pallas import tpu_sc as plsc

N = 10000
E = 320000
F = 128
R = 32
P = 4
FQ = F // P
GW = 6 * FQ
VW = 3 * FQ
GP = 128
WP = 96
VP = 128
NB = 1000
EB = 2000
NCORES = 2
NSUB = 16
NWORK = NCORES * NSUB
EPW = E // NWORK
C = 40
NCH = EPW // C
NP = 10240
RPT = NP // NSUB

_MASK_HI = -65536


def _pack16(lo, hi):
    lob = lax.bitcast_convert_type(lo, jnp.int32)
    hib = lax.bitcast_convert_type(hi, jnp.int32)
    return jnp.bitwise_or(
        lax.shift_right_logical(lob + 32768, 16),
        jnp.bitwise_and(hib + 32768, _MASK_HI))


def _phi_v_body(s_ref, v_ref, w1_ref, b1_ref, w2_ref, b2_ref, phi_ref,
                v4_ref):
    h = jax.nn.silu(s_ref[...] @ w1_ref[...] + b1_ref[...])
    phi = h @ w2_ref[...] + b2_ref[...]
    vblk = v_ref[...]
    pad_g = jnp.zeros((phi.shape[0], GP - GW // 2), jnp.int32)
    pad_v = jnp.zeros((phi.shape[0], VP - VW), jnp.float32)
    for p in range(P):
        blk = phi[:, p * GW:(p + 1) * GW]
        phi_ref[p] = jnp.concatenate(
            [_pack16(blk[:, :GW // 2], blk[:, GW // 2:]), pad_g], axis=1)
        v4_ref[p] = jnp.concatenate(
            [vblk[:, d * F + p * FQ: d * F + (p + 1) * FQ] for d in range(3)]
            + [pad_v], axis=1)


def _wfilt_body(r1_ref, r2_ref, f1_ref, f2_ref, wr_ref, br_ref, wf_ref):
    f1 = f1_ref[...]
    f2 = f2_ref[...]
    rs = r1_ref[...] * f1 + r2_ref[...] * f2
    wf = rs @ wr_ref[...] + br_ref[...] * (f1 + f2)
    for p in range(P):
        blk = wf[:, p * GW:(p + 1) * GW]
        wf_ref[p] = _pack16(blk[:, :GW // 2], blk[:, GW // 2:])


def _combine_body(s_ref, v_ref, p0_ref, p1_ref, p2_ref, p3_ref,
                  os_ref, ov_ref):
    parts = [p0_ref, p1_ref, p2_ref, p3_ref]
    ps = [pr[0] + pr[1] for pr in parts]
    os_ref[...] = s_ref[...] + jnp.concatenate(
        [ps[p][:, 0:FQ] for p in range(P)], axis=1)
    ov_ref[...] = v_ref[...] + jnp.concatenate(
        [ps[p][:, FQ * (d + 1): FQ * (d + 2)] for d in range(3)
         for p in range(P)], axis=1)


def _scatter_add_rows(src, acc, idx):
    pltpu.sync_copy(src, acc.at[idx], add=True)


def _halves(buf, e, g):
    r = buf[e, pl.ds(g * 16, 16)]
    lo = plsc.bitcast(lax.shift_left(r, jnp.full((16,), 16, jnp.int32)),
                      jnp.float32)
    hi = plsc.bitcast(lax.bitwise_and(r, jnp.full((16,), _MASK_HI,
                                                  jnp.int32)),
                      jnp.float32)
    return lo, hi


def _sc_body(*refs):
    (phi_refs, v_refs, wf_refs) = (refs[0:P], refs[P:2 * P], refs[2 * P:3 * P])
    (idxi_hbm, idxj_hbm, uv1_hbm, uv2_hbm, zeros_hbm) = refs[3 * P:3 * P + 5]
    out_refs = refs[3 * P + 5:4 * P + 5]
    sc = refs[4 * P + 5:]
    idxj = sc[0:2]
    idxi = sc[2:4]
    phib = sc[4:6]
    wfb = sc[6:8]
    vb = sc[8:10]
    uv1b = sc[10:12]
    uv2b = sc[12:14]
    outb = sc[14]
    acc = sc[15]
    sem_l = sc[16:18]
    sem_g = sc[18:20]

    cid = lax.axis_index("c")
    sid = lax.axis_index("s")
    wid = sid * NCORES + cid
    row0 = sid * RPT

    def make_edge(phib_, wfb_, vb_, uv1b_, uv2b_):
        def edge(e, carry):
            u1w = uv1b_[pl.ds(3 * e, 16)]
            u2w = uv2b_[pl.ds(3 * e, 16)]

            def bcast(win, d):
                dn = lax.GatherDimensionNumbers(
                    offset_dims=(), collapsed_slice_dims=(0,),
                    start_index_map=(0,))
                return lax.gather(
                    win, jnp.full((16, 1), d, jnp.int32), dn,
                    slice_sizes=(1,),
                    mode=lax.GatherScatterMode.PROMISE_IN_BOUNDS)
            u1 = [bcast(u1w, d) for d in range(3)]
            u2 = [bcast(u2w, d) for d in range(3)]
            x = {}
            for g in range(6):
                plo, phi_ = _halves(phib_, e, g)
                wlo, whi = _halves(wfb_, e, g)
                x[(g, 0)] = plo * wlo
                x[(g, 1)] = phi_ * whi
            for k in range(2):
                outb[e, pl.ds(k * 16, 16)] = x[(0, k)]
            vj = {}
            for d in range(3):
                for k in range(2):
                    vj[(d, k)] = vb_[e, pl.ds(d * FQ + k * 16, 16)]
            t = {}
            for d in range(3):
                for k in range(2):
                    t[(d, k)] = x[(4, k)] * u1[d] + x[(5, k)] * u2[d]
            for d in range(3):
                a = (d + 1) % 3
                b = (d + 2) % 3
                for k in range(2):
                    xv = (vj[(d, k)] * x[(1, k)] + x[(2, k)] * u1[d]
                          + x[(3, k)] * u2[d]
                          + vj[(a, k)] * t[(b, k)] - vj[(b, k)] * t[(a, k)])
                    outb[e, pl.ds(FQ + d * FQ + k * 16, 16)] = xv
            return carry
        return edge

    for p in range(P):
        pltpu.sync_copy(zeros_hbm.at[pl.ds(row0, RPT)],
                        acc.at[pl.ds(row0, RPT)])
        plsc.subcore_barrier()

        def load_pairs(c, B):
            e0 = wid * EPW + c * C
            return [
                (idxj_hbm.at[pl.ds(e0, C)], idxj[B]),
                (idxi_hbm.at[pl.ds(e0, C)], idxi[B]),
                (wf_refs[p].at[pl.ds(e0, C)], wfb[B]),
                (uv1_hbm.at[pl.ds(3 * e0, 3 * C)], uv1b[B].at[pl.ds(0, 3 * C)]),
                (uv2_hbm.at[pl.ds(3 * e0, 3 * C)], uv2b[B].at[pl.ds(0, 3 * C)]),
            ]

        def issue_loads(c, B):
            for src, dst in load_pairs(c, B):
                pltpu.async_copy(src, dst, sem_l[B])

        def wait_loads(c, B):
            for src, dst in load_pairs(c, B):
                pltpu.make_async_copy(src, dst, sem_l[B]).wait()

        def issue_gathers(B):
            pltpu.async_copy(phi_refs[p].at[idxj[B]], phib[B], sem_g[B])
            pltpu.async_copy(v_refs[p].at[idxj[B]], vb[B], sem_g[B])

        def wait_gathers(B):
            pltpu.make_async_copy(
                phi_refs[p].at[idxj[B]], phib[B], sem_g[B]).wait()
            pltpu.make_async_copy(
                v_refs[p].at[idxj[B]], vb[B], sem_g[B]).wait()

        def compute(B):
            lax.fori_loop(
                0, C,
                make_edge(phib[B], wfb[B], vb[B], uv1b[B], uv2b[B]),
                0, unroll=4)
            _scatter_add_rows(outb, acc, idxi[B])

        issue_loads(0, 0)
        wait_loads(0, 0)
        issue_gathers(0)
        issue_loads(1, 1)

        def pair(i, carry):
            c0 = 2 * i
            wait_loads(c0 + 1, 1)
            issue_gathers(1)
            wait_gathers(0)
            compute(0)
            issue_loads(c0 + 2, 0)
            wait_gathers(1)
            compute(1)
            issue_loads(c0 + 3, 1)
            wait_loads(c0 + 2, 0)
            issue_gathers(0)
            return carry

        lax.fori_loop(0, NCH // 2 - 1, pair, 0)
        wait_loads(NCH - 1, 1)
        issue_gathers(1)
        wait_gathers(0)
        compute(0)
        wait_gathers(1)
        compute(1)

        plsc.subcore_barrier()
        pltpu.sync_copy(acc.at[pl.ds(row0, RPT)],
                        out_refs[p].at[cid, pl.ds(row0, RPT)])


def _make_sc_kernel():
    mesh = plsc.VectorSubcoreMesh(core_axis_name="c", subcore_axis_name="s",
                                  num_cores=NCORES, num_subcores=NSUB)
    out_type = [jax.ShapeDtypeStruct((NCORES, NP, F), jnp.float32)
                for _ in range(P)]
    scratch = (
        [pltpu.VMEM((C,), jnp.int32) for _ in range(2)]
        + [pltpu.VMEM((C,), jnp.int32) for _ in range(2)]
        + [pltpu.VMEM((C, GP), jnp.int32) for _ in range(2)]
        + [pltpu.VMEM((C, WP), jnp.int32) for _ in range(2)]
        + [pltpu.VMEM((C, VP), jnp.float32) for _ in range(2)]
        + [pltpu.VMEM((3 * C + 16,), jnp.float32) for _ in range(2)]
        + [pltpu.VMEM((3 * C + 16,), jnp.float32) for _ in range(2)]
        + [
            pltpu.VMEM((C, F), jnp.float32),
            pltpu.VMEM_SHARED((NP, F), jnp.float32),
            pltpu.SemaphoreType.DMA,
            pltpu.SemaphoreType.DMA,
            pltpu.SemaphoreType.DMA,
            pltpu.SemaphoreType.DMA,
        ])
    return pl.kernel(
        _sc_body, out_type=out_type, mesh=mesh, scratch_types=scratch,
        compiler_params=pltpu.CompilerParams(needs_layout_passes=False))


def _packed_perm():
    perm = (np.arange(6)[None, None, :, None] * F
            + np.arange(P)[:, None, None, None] * FQ
            + np.arange(2)[None, :, None, None] * 16
            + np.arange(16)[None, None, None, :])
    return perm.reshape(-1).astype(np.int32)


def kernel(s, v, radial_embeddings_1, radial_embeddings_2, f_cut_1, f_cut_2,
           unit_vectors_1, unit_vectors_2, edge_index, W1, b1, W2, b2, Wr,
           br):
    s2 = s.reshape(N, F)
    v2 = v.reshape(N, 3 * F)
    r1 = radial_embeddings_1.reshape(E, R)
    r2 = radial_embeddings_2.reshape(E, R)
    f1 = f_cut_1.reshape(E, 1)
    f2 = f_cut_2.reshape(E, 1)
    uv1 = unit_vectors_1.reshape(3 * E)
    uv2 = unit_vectors_2.reshape(3 * E)
    ei = edge_index.astype(jnp.int32)
    idx_i = ei[0]
    idx_j = ei[1]
    perm = _packed_perm()
    W2p = W2[:, perm]
    b2p = b2[perm].reshape(1, 6 * F)
    Wrp = Wr[:, perm]
    brp = br[perm].reshape(1, 6 * F)
    b1r = b1.reshape(1, F)
    zeros = jnp.zeros((NP, F), jnp.float32)

    phi4, v4 = pl.pallas_call(
        _phi_v_body,
        grid=(N // NB,),
        in_specs=[
            pl.BlockSpec((NB, F), lambda i: (i, 0)),
            pl.BlockSpec((NB, 3 * F), lambda i: (i, 0)),
            pl.BlockSpec((F, F), lambda i: (0, 0)),
            pl.BlockSpec((1, F), lambda i: (0, 0)),
            pl.BlockSpec((F, 6 * F), lambda i: (0, 0)),
            pl.BlockSpec((1, 6 * F), lambda i: (0, 0)),
        ],
        out_specs=[
            pl.BlockSpec((P, NB, GP), lambda i: (0, i, 0)),
            pl.BlockSpec((P, NB, VP), lambda i: (0, i, 0)),
        ],
        out_shape=[
            jax.ShapeDtypeStruct((P, N, GP), jnp.int32),
            jax.ShapeDtypeStruct((P, N, VP), jnp.float32),
        ],
    )(s2, v2, W1, b1r, W2p, b2p)

    wf4 = pl.pallas_call(
        _wfilt_body,
        grid=(E // EB,),
        in_specs=[
            pl.BlockSpec((EB, R), lambda i: (i, 0)),
            pl.BlockSpec((EB, R), lambda i: (i, 0)),
            pl.BlockSpec((EB, 1), lambda i: (i, 0)),
            pl.BlockSpec((EB, 1), lambda i: (i, 0)),
            pl.BlockSpec((R, 6 * F), lambda i: (0, 0)),
            pl.BlockSpec((1, 6 * F), lambda i: (0, 0)),
        ],
        out_specs=[pl.BlockSpec((P, EB, WP), lambda i: (0, i, 0))],
        out_shape=[jax.ShapeDtypeStruct((P, E, WP), jnp.int32)],
    )(r1, r2, f1, f2, Wrp, brp)
    wf4 = wf4[0]

    sc = _make_sc_kernel()
    parts = sc(phi4[0], phi4[1], phi4[2], phi4[3],
               v4[0], v4[1], v4[2], v4[3],
               wf4[0], wf4[1], wf4[2], wf4[3],
               idx_i, idx_j, uv1, uv2, zeros)

    out_s, out_v = pl.pallas_call(
        _combine_body,
        grid=(N // NB,),
        in_specs=[
            pl.BlockSpec((NB, F), lambda i: (i, 0)),
            pl.BlockSpec((NB, 3 * F), lambda i: (i, 0)),
        ] + [pl.BlockSpec((NCORES, NB, F), lambda i: (0, i, 0))] * P,
        out_specs=[
            pl.BlockSpec((NB, F), lambda i: (i, 0)),
            pl.BlockSpec((NB, 3 * F), lambda i: (i, 0)),
        ],
        out_shape=[
            jax.ShapeDtypeStruct((N, F), jnp.float32),
            jax.ShapeDtypeStruct((N, 3 * F), jnp.float32),
        ],
    )(s2, v2, parts[0], parts[1], parts[2], parts[3])

    return (out_s.reshape(N, 1, F), out_v.reshape(N, 3, F))


# speedup vs baseline: 12.8989x; 1.1243x over previous
import functools

import jax
import jax.numpy as jnp
import numpy as np
from jax import lax
from jax.experimental import pallas as pl
from jax.experimental.pallas import tpu as pltpu
from jax.experimental.pallas import tpu_sc as plsc

N = 10000
E = 320000
F = 128
R = 32
P = 4
FQ = F // P
GW = 6 * FQ
VW = 3 * FQ
GP = 128
WP = 96
VP = 128
NB = 1000
EB = 2000
NCORES = 2
NSUB = 16
NWORK = NCORES * NSUB
EPW = E // NWORK
C = 40
NCH = EPW // C
NP = 10240
RPT = NP // NSUB

_MASK_HI = -65536


def _pack16(lo, hi):
    lob = lax.bitcast_convert_type(lo, jnp.int32)
    hib = lax.bitcast_convert_type(hi, jnp.int32)
    return jnp.bitwise_or(
        lax.shift_right_logical(lob + 32768, 16),
        jnp.bitwise_and(hib + 32768, _MASK_HI))


def _phi_v_body(s_ref, v_ref, w1_ref, b1_ref, w2_ref, b2_ref, phi_ref,
                v4_ref):
    h = jax.nn.silu(s_ref[...] @ w1_ref[...] + b1_ref[...])
    phi = h @ w2_ref[...] + b2_ref[...]
    vblk = v_ref[...]
    pad_g = jnp.zeros((phi.shape[0], GP - GW // 2), jnp.int32)
    pad_v = jnp.zeros((phi.shape[0], VP - VW), jnp.float32)
    for p in range(P):
        blk = phi[:, p * GW:(p + 1) * GW]
        phi_ref[p] = jnp.concatenate(
            [_pack16(blk[:, :GW // 2], blk[:, GW // 2:]), pad_g], axis=1)
        v4_ref[p] = jnp.concatenate(
            [vblk[:, d * F + p * FQ: d * F + (p + 1) * FQ] for d in range(3)]
            + [pad_v], axis=1)


def _wfilt_body(r1_ref, r2_ref, f1_ref, f2_ref, wr_ref, br_ref, wf_ref):
    f1 = f1_ref[...]
    f2 = f2_ref[...]
    rs = r1_ref[...] * f1 + r2_ref[...] * f2
    wf = rs @ wr_ref[...] + br_ref[...] * (f1 + f2)
    for p in range(P):
        blk = wf[:, p * GW:(p + 1) * GW]
        wf_ref[p] = _pack16(blk[:, :GW // 2], blk[:, GW // 2:])


def _combine_body(s_ref, v_ref, parts_ref, os_ref, ov_ref):
    ps = [parts_ref[p, 0] + parts_ref[p, 1] for p in range(P)]
    os_ref[...] = s_ref[...] + jnp.concatenate(
        [ps[p][:, 0:FQ] for p in range(P)], axis=1)
    ov_ref[...] = v_ref[...] + jnp.concatenate(
        [ps[p][:, FQ * (d + 1): FQ * (d + 2)] for d in range(3)
         for p in range(P)], axis=1)


def _scatter_add_rows(src, acc, idx):
    pltpu.sync_copy(src, acc.at[idx], add=True)


def _halves(buf, e, g):
    r = buf[e, pl.ds(g * 16, 16)]
    lo = plsc.bitcast(lax.shift_left(r, jnp.full((16,), 16, jnp.int32)),
                      jnp.float32)
    hi = plsc.bitcast(lax.bitwise_and(r, jnp.full((16,), _MASK_HI,
                                                  jnp.int32)),
                      jnp.float32)
    return lo, hi


def _sc_body(*refs):
    (phi_hbm, v_hbm, wf_hbm, idxi_hbm, idxj_hbm, uv1_hbm, uv2_hbm,
     out_hbm) = refs[0:8]
    sc = refs[8:]
    idxj = sc[0:2]
    idxi = sc[2:4]
    phib = sc[4:6]
    wfb = sc[6:8]
    vb = sc[8:10]
    uv1b = sc[10:12]
    uv2b = sc[12:14]
    outb = sc[14]
    acc = sc[15]
    sem_l = sc[16:18]
    sem_g = sc[18:20]

    cid = lax.axis_index("c")
    sid = lax.axis_index("s")
    wid = sid * NCORES + cid
    row0 = sid * RPT

    def make_edge(phib_, wfb_, vb_, uv1b_, uv2b_):
        def edge(e, carry):
            u1w = uv1b_[pl.ds(3 * e, 16)]
            u2w = uv2b_[pl.ds(3 * e, 16)]

            def bcast(win, d):
                dn = lax.GatherDimensionNumbers(
                    offset_dims=(), collapsed_slice_dims=(0,),
                    start_index_map=(0,))
                return lax.gather(
                    win, jnp.full((16, 1), d, jnp.int32), dn,
                    slice_sizes=(1,),
                    mode=lax.GatherScatterMode.PROMISE_IN_BOUNDS)
            u1 = [bcast(u1w, d) for d in range(3)]
            u2 = [bcast(u2w, d) for d in range(3)]
            x = {}
            for g in range(6):
                plo, phi_ = _halves(phib_, e, g)
                wlo, whi = _halves(wfb_, e, g)
                x[(g, 0)] = plo * wlo
                x[(g, 1)] = phi_ * whi
            for k in range(2):
                outb[e, pl.ds(k * 16, 16)] = x[(0, k)]
            vj = {}
            for d in range(3):
                for k in range(2):
                    vj[(d, k)] = vb_[e, pl.ds(d * FQ + k * 16, 16)]
            t = {}
            for d in range(3):
                for k in range(2):
                    t[(d, k)] = x[(4, k)] * u1[d] + x[(5, k)] * u2[d]
            for d in range(3):
                a = (d + 1) % 3
                b = (d + 2) % 3
                for k in range(2):
                    xv = (vj[(d, k)] * x[(1, k)] + x[(2, k)] * u1[d]
                          + x[(3, k)] * u2[d]
                          + vj[(a, k)] * t[(b, k)] - vj[(b, k)] * t[(a, k)])
                    outb[e, pl.ds(FQ + d * FQ + k * 16, 16)] = xv
            return carry
        return edge

    def zero_outb():
        zv = jnp.zeros((16,), jnp.float32)

        def zrow(e, carry):
            for k in range(F // 16):
                outb[e, pl.ds(k * 16, 16)] = zv
            return carry
        lax.fori_loop(0, C, zrow, 0, unroll=4)

    for p in range(P):
        zero_outb()
        for k in range(RPT // C):
            pltpu.sync_copy(outb, acc.at[pl.ds(row0 + k * C, C)])
        plsc.subcore_barrier()

        def load_pairs(c, B):
            e0 = wid * EPW + c * C
            return [
                (idxj_hbm.at[pl.ds(e0, C)], idxj[B]),
                (idxi_hbm.at[pl.ds(e0, C)], idxi[B]),
                (wf_hbm.at[pl.ds(p * E + e0, C)], wfb[B]),
                (uv1_hbm.at[pl.ds(3 * e0, 3 * C)], uv1b[B].at[pl.ds(0, 3 * C)]),
                (uv2_hbm.at[pl.ds(3 * e0, 3 * C)], uv2b[B].at[pl.ds(0, 3 * C)]),
            ]

        def issue_loads(c, B):
            for src, dst in load_pairs(c, B):
                pltpu.async_copy(src, dst, sem_l[B])

        def wait_loads(c, B):
            for src, dst in load_pairs(c, B):
                pltpu.make_async_copy(src, dst, sem_l[B]).wait()

        phi_p = phi_hbm.at[pl.ds(p * N, N)]
        v_p = v_hbm.at[pl.ds(p * N, N)]

        def issue_gathers(B):
            pltpu.async_copy(phi_p.at[idxj[B]], phib[B], sem_g[B])
            pltpu.async_copy(v_p.at[idxj[B]], vb[B], sem_g[B])

        def wait_gathers(B):
            pltpu.make_async_copy(
                phi_p.at[idxj[B]], phib[B], sem_g[B]).wait()
            pltpu.make_async_copy(
                v_p.at[idxj[B]], vb[B], sem_g[B]).wait()

        def compute(B):
            lax.fori_loop(
                0, C,
                make_edge(phib[B], wfb[B], vb[B], uv1b[B], uv2b[B]),
                0, unroll=4)
            _scatter_add_rows(outb, acc, idxi[B])

        issue_loads(0, 0)
        wait_loads(0, 0)
        issue_gathers(0)
        issue_loads(1, 1)

        def pair(i, carry):
            c0 = 2 * i
            wait_loads(c0 + 1, 1)
            issue_gathers(1)
            wait_gathers(0)
            compute(0)
            issue_loads(c0 + 2, 0)
            wait_gathers(1)
            compute(1)
            issue_loads(c0 + 3, 1)
            wait_loads(c0 + 2, 0)
            issue_gathers(0)
            return carry

        lax.fori_loop(0, NCH // 2 - 1, pair, 0)
        wait_loads(NCH - 1, 1)
        issue_gathers(1)
        wait_gathers(0)
        compute(0)
        wait_gathers(1)
        compute(1)

        plsc.subcore_barrier()
        pltpu.sync_copy(
            acc.at[pl.ds(row0, RPT)],
            out_hbm.at[pl.ds((p * NCORES + cid) * NP + row0, RPT)])


def _make_sc_kernel():
    mesh = plsc.VectorSubcoreMesh(core_axis_name="c", subcore_axis_name="s",
                                  num_cores=NCORES, num_subcores=NSUB)
    out_type = jax.ShapeDtypeStruct((P * NCORES * NP, F), jnp.float32)
    scratch = (
        [pltpu.VMEM((C,), jnp.int32) for _ in range(2)]
        + [pltpu.VMEM((C,), jnp.int32) for _ in range(2)]
        + [pltpu.VMEM((C, GP), jnp.int32) for _ in range(2)]
        + [pltpu.VMEM((C, WP), jnp.int32) for _ in range(2)]
        + [pltpu.VMEM((C, VP), jnp.float32) for _ in range(2)]
        + [pltpu.VMEM((3 * C + 16,), jnp.float32) for _ in range(2)]
        + [pltpu.VMEM((3 * C + 16,), jnp.float32) for _ in range(2)]
        + [
            pltpu.VMEM((C, F), jnp.float32),
            pltpu.VMEM_SHARED((NP, F), jnp.float32),
            pltpu.SemaphoreType.DMA,
            pltpu.SemaphoreType.DMA,
            pltpu.SemaphoreType.DMA,
            pltpu.SemaphoreType.DMA,
        ])
    return pl.kernel(
        _sc_body, out_type=out_type, mesh=mesh, scratch_types=scratch,
        compiler_params=pltpu.CompilerParams(needs_layout_passes=False))


def _packed_perm():
    perm = (np.arange(6)[None, None, :, None] * F
            + np.arange(P)[:, None, None, None] * FQ
            + np.arange(2)[None, :, None, None] * 16
            + np.arange(16)[None, None, None, :])
    return perm.reshape(-1).astype(np.int32)


def kernel(s, v, radial_embeddings_1, radial_embeddings_2, f_cut_1, f_cut_2,
           unit_vectors_1, unit_vectors_2, edge_index, W1, b1, W2, b2, Wr,
           br):
    s2 = s.reshape(N, F)
    v2 = v.reshape(N, 3 * F)
    r1 = radial_embeddings_1.reshape(E, R)
    r2 = radial_embeddings_2.reshape(E, R)
    f1 = f_cut_1.reshape(E, 1)
    f2 = f_cut_2.reshape(E, 1)
    uv1 = unit_vectors_1.reshape(3 * E)
    uv2 = unit_vectors_2.reshape(3 * E)
    ei = edge_index.astype(jnp.int32)
    idx_i = ei[0]
    idx_j = ei[1]
    perm = _packed_perm()
    W2p = W2[:, perm]
    b2p = b2[perm].reshape(1, 6 * F)
    Wrp = Wr[:, perm]
    brp = br[perm].reshape(1, 6 * F)
    b1r = b1.reshape(1, F)

    phi4, v4 = pl.pallas_call(
        _phi_v_body,
        grid=(N // NB,),
        in_specs=[
            pl.BlockSpec((NB, F), lambda i: (i, 0)),
            pl.BlockSpec((NB, 3 * F), lambda i: (i, 0)),
            pl.BlockSpec((F, F), lambda i: (0, 0)),
            pl.BlockSpec((1, F), lambda i: (0, 0)),
            pl.BlockSpec((F, 6 * F), lambda i: (0, 0)),
            pl.BlockSpec((1, 6 * F), lambda i: (0, 0)),
        ],
        out_specs=[
            pl.BlockSpec((P, NB, GP), lambda i: (0, i, 0)),
            pl.BlockSpec((P, NB, VP), lambda i: (0, i, 0)),
        ],
        out_shape=[
            jax.ShapeDtypeStruct((P, N, GP), jnp.int32),
            jax.ShapeDtypeStruct((P, N, VP), jnp.float32),
        ],
    )(s2, v2, W1, b1r, W2p, b2p)

    wf4 = pl.pallas_call(
        _wfilt_body,
        grid=(E // EB,),
        in_specs=[
            pl.BlockSpec((EB, R), lambda i: (i, 0)),
            pl.BlockSpec((EB, R), lambda i: (i, 0)),
            pl.BlockSpec((EB, 1), lambda i: (i, 0)),
            pl.BlockSpec((EB, 1), lambda i: (i, 0)),
            pl.BlockSpec((R, 6 * F), lambda i: (0, 0)),
            pl.BlockSpec((1, 6 * F), lambda i: (0, 0)),
        ],
        out_specs=[pl.BlockSpec((P, EB, WP), lambda i: (0, i, 0))],
        out_shape=[jax.ShapeDtypeStruct((P, E, WP), jnp.int32)],
    )(r1, r2, f1, f2, Wrp, brp)
    wf4 = wf4[0]

    sc = _make_sc_kernel()
    parts = sc(phi4.reshape(P * N, GP), v4.reshape(P * N, VP),
               wf4.reshape(P * E, WP), idx_i, idx_j, uv1, uv2)
    parts = parts.reshape(P, NCORES, NP, F)

    out_s, out_v = pl.pallas_call(
        _combine_body,
        grid=(N // NB,),
        in_specs=[
            pl.BlockSpec((NB, F), lambda i: (i, 0)),
            pl.BlockSpec((NB, 3 * F), lambda i: (i, 0)),
            pl.BlockSpec((P, NCORES, NB, F), lambda i: (0, 0, i, 0)),
        ],
        out_specs=[
            pl.BlockSpec((NB, F), lambda i: (i, 0)),
            pl.BlockSpec((NB, 3 * F), lambda i: (i, 0)),
        ],
        out_shape=[
            jax.ShapeDtypeStruct((N, F), jnp.float32),
            jax.ShapeDtypeStruct((N, 3 * F), jnp.float32),
        ],
    )(s2, v2, parts)

    return (out_s.reshape(N, 1, F), out_v.reshape(N, 3, F))


# speedup vs baseline: 12.9080x; 1.0007x over previous
import functools

import jax
import jax.numpy as jnp
import numpy as np
from jax import lax
from jax.experimental import pallas as pl
from jax.experimental.pallas import tpu as pltpu
from jax.experimental.pallas import tpu_sc as plsc

N = 10000
E = 320000
F = 128
R = 32
P = 4
FQ = F // P
GW = 6 * FQ
VW = 3 * FQ
GP = 128
WP = 96
VP = 128
NB = 1000
EB = 2000
NCORES = 2
NSUB = 16
NWORK = NCORES * NSUB
EPW = E // NWORK
C = 40
NCH = EPW // C
NP = 10240
RPT = NP // NSUB

_MASK_HI = -65536


def _pack16(lo, hi):
    lob = lax.bitcast_convert_type(lo, jnp.int32)
    hib = lax.bitcast_convert_type(hi, jnp.int32)
    return jnp.bitwise_or(
        lax.shift_right_logical(lob + 32768, 16),
        jnp.bitwise_and(hib + 32768, _MASK_HI))


def _phi_v_body(s_ref, v_ref, w1_ref, b1_ref, w2_ref, b2_ref, phi_ref,
                v4_ref):
    h = jax.nn.silu(s_ref[...] @ w1_ref[...] + b1_ref[...])
    phi = h @ w2_ref[...] + b2_ref[...]
    vblk = v_ref[...]
    pad_g = jnp.zeros((phi.shape[0], GP - GW // 2), jnp.int32)
    pad_v = jnp.zeros((phi.shape[0], VP - VW), jnp.float32)
    for p in range(P):
        blk = phi[:, p * GW:(p + 1) * GW]
        phi_ref[p] = jnp.concatenate(
            [_pack16(blk[:, :GW // 2], blk[:, GW // 2:]), pad_g], axis=1)
        v4_ref[p] = jnp.concatenate(
            [vblk[:, d * F + p * FQ: d * F + (p + 1) * FQ] for d in range(3)]
            + [pad_v], axis=1)


def _wfilt_body(r1_ref, r2_ref, f1_ref, f2_ref, wr_ref, br_ref, wf_ref):
    f1 = f1_ref[...]
    f2 = f2_ref[...]
    rs = r1_ref[...] * f1 + r2_ref[...] * f2
    wf = rs @ wr_ref[...] + br_ref[...] * (f1 + f2)
    for p in range(P):
        blk = wf[:, p * GW:(p + 1) * GW]
        wf_ref[p] = _pack16(blk[:, :GW // 2], blk[:, GW // 2:])


def _combine_body(s_ref, v_ref, parts_ref, os_ref, ov_ref):
    ps = [parts_ref[p, 0] + parts_ref[p, 1] for p in range(P)]
    os_ref[...] = s_ref[...] + jnp.concatenate(
        [ps[p][:, 0:FQ] for p in range(P)], axis=1)
    ov_ref[...] = v_ref[...] + jnp.concatenate(
        [ps[p][:, FQ * (d + 1): FQ * (d + 2)] for d in range(3)
         for p in range(P)], axis=1)


def _scatter_add_rows(src, acc, idx):
    pltpu.sync_copy(src, acc.at[idx], add=True)


def _halves(buf, e, g):
    r = buf[e, pl.ds(g * 16, 16)]
    lo = plsc.bitcast(lax.shift_left(r, jnp.full((16,), 16, jnp.int32)),
                      jnp.float32)
    hi = plsc.bitcast(lax.bitwise_and(r, jnp.full((16,), _MASK_HI,
                                                  jnp.int32)),
                      jnp.float32)
    return lo, hi


def _sc_body(*refs):
    (phi_hbm, v_hbm, wf_hbm, idxi_hbm, idxj_hbm, uv1_hbm, uv2_hbm,
     out_hbm) = refs[0:8]
    sc = refs[8:]
    idxj = sc[0:2]
    idxi = sc[2:4]
    phib = sc[4:6]
    wfb = sc[6:8]
    vb = sc[8:10]
    uv1b = sc[10:12]
    uv2b = sc[12:14]
    outb = sc[14]
    acc = sc[15]
    sem_l = sc[16:18]
    sem_g = sc[18:20]

    cid = lax.axis_index("c")
    sid = lax.axis_index("s")
    wid = sid * NCORES + cid
    row0 = sid * RPT

    def make_edge(phib_, wfb_, vb_, uv1b_, uv2b_):
        def edge(e, carry):
            u1w = uv1b_[pl.ds(3 * e, 16)]
            u2w = uv2b_[pl.ds(3 * e, 16)]

            def bcast(win, d):
                dn = lax.GatherDimensionNumbers(
                    offset_dims=(), collapsed_slice_dims=(0,),
                    start_index_map=(0,))
                return lax.gather(
                    win, jnp.full((16, 1), d, jnp.int32), dn,
                    slice_sizes=(1,),
                    mode=lax.GatherScatterMode.PROMISE_IN_BOUNDS)
            u1 = [bcast(u1w, d) for d in range(3)]
            u2 = [bcast(u2w, d) for d in range(3)]
            x = {}
            for g in range(6):
                plo, phi_ = _halves(phib_, e, g)
                wlo, whi = _halves(wfb_, e, g)
                x[(g, 0)] = plo * wlo
                x[(g, 1)] = phi_ * whi
            for k in range(2):
                outb[e, pl.ds(k * 16, 16)] = x[(0, k)]
            vj = {}
            for d in range(3):
                for k in range(2):
                    vj[(d, k)] = vb_[e, pl.ds(d * FQ + k * 16, 16)]
            t = {}
            for d in range(3):
                for k in range(2):
                    t[(d, k)] = x[(4, k)] * u1[d] + x[(5, k)] * u2[d]
            for d in range(3):
                a = (d + 1) % 3
                b = (d + 2) % 3
                for k in range(2):
                    xv = (vj[(d, k)] * x[(1, k)] + x[(2, k)] * u1[d]
                          + x[(3, k)] * u2[d]
                          + vj[(a, k)] * t[(b, k)] - vj[(b, k)] * t[(a, k)])
                    outb[e, pl.ds(FQ + d * FQ + k * 16, 16)] = xv
            return carry
        return edge

    def zero_outb():
        zv = jnp.zeros((16,), jnp.float32)

        def zrow(e, carry):
            for k in range(F // 16):
                outb[e, pl.ds(k * 16, 16)] = zv
            return carry
        lax.fori_loop(0, C, zrow, 0, unroll=4)

    for p in range(P):
        zero_outb()
        for k in range(RPT // C):
            pltpu.sync_copy(outb, acc.at[pl.ds(row0 + k * C, C)])
        plsc.subcore_barrier()

        def load_pairs(c, B):
            e0 = wid * EPW + c * C
            return [
                (idxj_hbm.at[pl.ds(e0, C)], idxj[B]),
                (idxi_hbm.at[pl.ds(e0, C)], idxi[B]),
                (wf_hbm.at[pl.ds(p * E + e0, C)], wfb[B]),
                (uv1_hbm.at[pl.ds(3 * e0, 3 * C)], uv1b[B].at[pl.ds(0, 3 * C)]),
                (uv2_hbm.at[pl.ds(3 * e0, 3 * C)], uv2b[B].at[pl.ds(0, 3 * C)]),
            ]

        def issue_loads(c, B):
            for src, dst in load_pairs(c, B):
                pltpu.async_copy(src, dst, sem_l[B])

        def wait_loads(c, B):
            for src, dst in load_pairs(c, B):
                pltpu.make_async_copy(src, dst, sem_l[B]).wait()

        phi_p = phi_hbm.at[pl.ds(p * N, N)]
        v_p = v_hbm.at[pl.ds(p * N, N)]

        def issue_gathers(B):
            pltpu.async_copy(phi_p.at[idxj[B]], phib[B], sem_g[B])
            pltpu.async_copy(v_p.at[idxj[B]], vb[B], sem_g[B])

        def wait_gathers(B):
            pltpu.make_async_copy(
                phi_p.at[idxj[B]], phib[B], sem_g[B]).wait()
            pltpu.make_async_copy(
                v_p.at[idxj[B]], vb[B], sem_g[B]).wait()

        def compute(B):
            lax.fori_loop(
                0, C,
                make_edge(phib[B], wfb[B], vb[B], uv1b[B], uv2b[B]),
                0, unroll=4)
            _scatter_add_rows(outb, acc, idxi[B])

        issue_loads(0, 0)
        wait_loads(0, 0)
        issue_gathers(0)
        issue_loads(1, 1)

        def pair(i, carry):
            c0 = 2 * i
            wait_loads(c0 + 1, 1)
            issue_gathers(1)
            wait_gathers(0)
            compute(0)
            issue_loads(c0 + 2, 0)
            wait_gathers(1)
            compute(1)
            issue_loads(c0 + 3, 1)
            wait_loads(c0 + 2, 0)
            issue_gathers(0)
            return carry

        lax.fori_loop(0, NCH // 2 - 1, pair, 0)
        wait_loads(NCH - 1, 1)
        issue_gathers(1)
        wait_gathers(0)
        compute(0)
        wait_gathers(1)
        compute(1)

        plsc.subcore_barrier()
        pltpu.sync_copy(
            acc.at[pl.ds(row0, RPT)],
            out_hbm.at[pl.ds((p * NCORES + cid) * NP + row0, RPT)])


def _make_sc_kernel():
    mesh = plsc.VectorSubcoreMesh(core_axis_name="c", subcore_axis_name="s",
                                  num_cores=NCORES, num_subcores=NSUB)
    out_type = jax.ShapeDtypeStruct((P * NCORES * NP, F), jnp.float32)
    scratch = (
        [pltpu.VMEM((C,), jnp.int32) for _ in range(2)]
        + [pltpu.VMEM((C,), jnp.int32) for _ in range(2)]
        + [pltpu.VMEM((C, GP), jnp.int32) for _ in range(2)]
        + [pltpu.VMEM((C, WP), jnp.int32) for _ in range(2)]
        + [pltpu.VMEM((C, VP), jnp.float32) for _ in range(2)]
        + [pltpu.VMEM((3 * C + 16,), jnp.float32) for _ in range(2)]
        + [pltpu.VMEM((3 * C + 16,), jnp.float32) for _ in range(2)]
        + [
            pltpu.VMEM((C, F), jnp.float32),
            pltpu.VMEM_SHARED((NP, F), jnp.float32),
            pltpu.SemaphoreType.DMA,
            pltpu.SemaphoreType.DMA,
            pltpu.SemaphoreType.DMA,
            pltpu.SemaphoreType.DMA,
        ])
    return pl.kernel(
        _sc_body, out_type=out_type, mesh=mesh, scratch_types=scratch,
        compiler_params=pltpu.CompilerParams(
            needs_layout_passes=False, disable_bounds_checks=True,
            disable_semaphore_checks=True, skip_device_barrier=True))


def _packed_perm():
    perm = (np.arange(6)[None, None, :, None] * F
            + np.arange(P)[:, None, None, None] * FQ
            + np.arange(2)[None, :, None, None] * 16
            + np.arange(16)[None, None, None, :])
    return perm.reshape(-1).astype(np.int32)


def kernel(s, v, radial_embeddings_1, radial_embeddings_2, f_cut_1, f_cut_2,
           unit_vectors_1, unit_vectors_2, edge_index, W1, b1, W2, b2, Wr,
           br):
    s2 = s.reshape(N, F)
    v2 = v.reshape(N, 3 * F)
    r1 = radial_embeddings_1.reshape(E, R)
    r2 = radial_embeddings_2.reshape(E, R)
    f1 = f_cut_1.reshape(E, 1)
    f2 = f_cut_2.reshape(E, 1)
    uv1 = unit_vectors_1.reshape(3 * E)
    uv2 = unit_vectors_2.reshape(3 * E)
    ei = edge_index.astype(jnp.int32)
    idx_i = ei[0]
    idx_j = ei[1]
    perm = _packed_perm()
    W2p = W2[:, perm]
    b2p = b2[perm].reshape(1, 6 * F)
    Wrp = Wr[:, perm]
    brp = br[perm].reshape(1, 6 * F)
    b1r = b1.reshape(1, F)

    phi4, v4 = pl.pallas_call(
        _phi_v_body,
        grid=(N // NB,),
        in_specs=[
            pl.BlockSpec((NB, F), lambda i: (i, 0)),
            pl.BlockSpec((NB, 3 * F), lambda i: (i, 0)),
            pl.BlockSpec((F, F), lambda i: (0, 0)),
            pl.BlockSpec((1, F), lambda i: (0, 0)),
            pl.BlockSpec((F, 6 * F), lambda i: (0, 0)),
            pl.BlockSpec((1, 6 * F), lambda i: (0, 0)),
        ],
        out_specs=[
            pl.BlockSpec((P, NB, GP), lambda i: (0, i, 0)),
            pl.BlockSpec((P, NB, VP), lambda i: (0, i, 0)),
        ],
        out_shape=[
            jax.ShapeDtypeStruct((P, N, GP), jnp.int32),
            jax.ShapeDtypeStruct((P, N, VP), jnp.float32),
        ],
    )(s2, v2, W1, b1r, W2p, b2p)

    wf4 = pl.pallas_call(
        _wfilt_body,
        grid=(E // EB,),
        in_specs=[
            pl.BlockSpec((EB, R), lambda i: (i, 0)),
            pl.BlockSpec((EB, R), lambda i: (i, 0)),
            pl.BlockSpec((EB, 1), lambda i: (i, 0)),
            pl.BlockSpec((EB, 1), lambda i: (i, 0)),
            pl.BlockSpec((R, 6 * F), lambda i: (0, 0)),
            pl.BlockSpec((1, 6 * F), lambda i: (0, 0)),
        ],
        out_specs=[pl.BlockSpec((P, EB, WP), lambda i: (0, i, 0))],
        out_shape=[jax.ShapeDtypeStruct((P, E, WP), jnp.int32)],
    )(r1, r2, f1, f2, Wrp, brp)
    wf4 = wf4[0]

    sc = _make_sc_kernel()
    parts = sc(phi4.reshape(P * N, GP), v4.reshape(P * N, VP),
               wf4.reshape(P * E, WP), idx_i, idx_j, uv1, uv2)
    parts = parts.reshape(P, NCORES, NP, F)

    out_s, out_v = pl.pallas_call(
        _combine_body,
        grid=(N // NB,),
        in_specs=[
            pl.BlockSpec((NB, F), lambda i: (i, 0)),
            pl.BlockSpec((NB, 3 * F), lambda i: (i, 0)),
            pl.BlockSpec((P, NCORES, NB, F), lambda i: (0, 0, i, 0)),
        ],
        out_specs=[
            pl.BlockSpec((NB, F), lambda i: (i, 0)),
            pl.BlockSpec((NB, 3 * F), lambda i: (i, 0)),
        ],
        out_shape=[
            jax.ShapeDtypeStruct((N, F), jnp.float32),
            jax.ShapeDtypeStruct((N, 3 * F), jnp.float32),
        ],
    )(s2, v2, parts)

    return (out_s.reshape(N, 1, F), out_v.reshape(N, 3, F))


# speedup vs baseline: 14.2060x; 1.1006x over previous
import functools

import jax
import jax.numpy as jnp
import numpy as np
from jax import lax
from jax.experimental import pallas as pl
from jax.experimental.pallas import tpu as pltpu
from jax.experimental.pallas import tpu_sc as plsc

N = 10000
E = 320000
F = 128
R = 32
P = 4
FQ = F // P
GW = 6 * FQ
VW = 3 * FQ
GP = 128
WP = 96
VP = 128
NB = 1000
EB = 2000
NCORES = 2
NSUB = 16
NWORK = NCORES * NSUB
EPW = E // NWORK
C = 40
NCH = EPW // C
NP = 10240
RPT = NP // NSUB

_MASK_HI = -65536


def _pack16(lo, hi):
    lob = lax.bitcast_convert_type(lo, jnp.int32)
    hib = lax.bitcast_convert_type(hi, jnp.int32)
    return jnp.bitwise_or(
        lax.shift_right_logical(lob + 32768, 16),
        jnp.bitwise_and(hib + 32768, _MASK_HI))


def _phi_v_body(s_ref, v_ref, w1_ref, b1_ref, w2_ref, b2_ref, phi_ref,
                v4_ref):
    h = jax.nn.silu(s_ref[...] @ w1_ref[...] + b1_ref[...])
    phi = h @ w2_ref[...] + b2_ref[...]
    vblk = v_ref[...]
    pad_g = jnp.zeros((phi.shape[0], GP - GW // 2), jnp.int32)
    pad_v = jnp.zeros((phi.shape[0], VP - VW), jnp.float32)
    for p in range(P):
        blk = phi[:, p * GW:(p + 1) * GW]
        phi_ref[p] = jnp.concatenate(
            [_pack16(blk[:, :GW // 2], blk[:, GW // 2:]), pad_g], axis=1)
        v4_ref[p] = jnp.concatenate(
            [vblk[:, d * F + p * FQ: d * F + (p + 1) * FQ] for d in range(3)]
            + [pad_v], axis=1)


def _wfilt_body(r1_ref, r2_ref, f1_ref, f2_ref, wr_ref, br_ref, wf_ref):
    f1 = f1_ref[...]
    f2 = f2_ref[...]
    rs = r1_ref[...] * f1 + r2_ref[...] * f2
    wf = rs @ wr_ref[...] + br_ref[...] * (f1 + f2)
    for p in range(P):
        blk = wf[:, p * GW:(p + 1) * GW]
        wf_ref[p] = _pack16(blk[:, :GW // 2], blk[:, GW // 2:])


def _combine_body(s_ref, v_ref, parts_ref, os_ref, ov_ref):
    ps = [parts_ref[p, 0] + parts_ref[p, 1] for p in range(P)]
    os_ref[...] = s_ref[...] + jnp.concatenate(
        [ps[p][:, 0:FQ] for p in range(P)], axis=1)
    ov_ref[...] = v_ref[...] + jnp.concatenate(
        [ps[p][:, FQ * (d + 1): FQ * (d + 2)] for d in range(3)
         for p in range(P)], axis=1)


def _scatter_add_rows(src, acc, idx):
    pltpu.sync_copy(src, acc.at[idx], add=True)


def _halves(buf, e, g):
    r = buf[e, pl.ds(g * 16, 16)]
    lo = plsc.bitcast(lax.shift_left(r, jnp.full((16,), 16, jnp.int32)),
                      jnp.float32)
    hi = plsc.bitcast(lax.bitwise_and(r, jnp.full((16,), _MASK_HI,
                                                  jnp.int32)),
                      jnp.float32)
    return lo, hi


def _sc_body(*refs):
    (phi_hbm, v_hbm, wf_hbm, idxi_hbm, idxj_hbm, uv1_hbm, uv2_hbm,
     out_hbm) = refs[0:8]
    sc = refs[8:]
    idxj = sc[0:2]
    idxi = sc[2:4]
    phib = sc[4:6]
    wfb = sc[6:8]
    vb = sc[8:10]
    uv1b = sc[10:12]
    uv2b = sc[12:14]
    outb = sc[14]
    acc = sc[15]
    sem_l = sc[16:18]
    sem_g = sc[18:20]

    cid = lax.axis_index("c")
    sid = lax.axis_index("s")
    wid = sid * NCORES + cid
    row0 = sid * RPT

    def make_edge(phib_, wfb_, vb_, uv1b_, uv2b_):
        def edge(e, carry):
            u1w = uv1b_[pl.ds(3 * e, 16)]
            u2w = uv2b_[pl.ds(3 * e, 16)]

            def bcast(win, d):
                dn = lax.GatherDimensionNumbers(
                    offset_dims=(), collapsed_slice_dims=(0,),
                    start_index_map=(0,))
                return lax.gather(
                    win, jnp.full((16, 1), d, jnp.int32), dn,
                    slice_sizes=(1,),
                    mode=lax.GatherScatterMode.PROMISE_IN_BOUNDS)
            u1 = [bcast(u1w, d) for d in range(3)]
            u2 = [bcast(u2w, d) for d in range(3)]
            x = {}
            for g in range(6):
                plo, phi_ = _halves(phib_, e, g)
                wlo, whi = _halves(wfb_, e, g)
                x[(g, 0)] = plo * wlo
                x[(g, 1)] = phi_ * whi
            for k in range(2):
                outb[e, pl.ds(k * 16, 16)] = x[(0, k)]
            vj = {}
            for d in range(3):
                for k in range(2):
                    vj[(d, k)] = vb_[e, pl.ds(d * FQ + k * 16, 16)]
            t = {}
            for d in range(3):
                for k in range(2):
                    t[(d, k)] = x[(4, k)] * u1[d] + x[(5, k)] * u2[d]
            for d in range(3):
                a = (d + 1) % 3
                b = (d + 2) % 3
                for k in range(2):
                    xv = (vj[(d, k)] * x[(1, k)] + x[(2, k)] * u1[d]
                          + x[(3, k)] * u2[d]
                          + vj[(a, k)] * t[(b, k)] - vj[(b, k)] * t[(a, k)])
                    outb[e, pl.ds(FQ + d * FQ + k * 16, 16)] = xv
            return carry
        return edge

    def zero_outb():
        zv = jnp.zeros((16,), jnp.float32)

        def zrow(e, carry):
            for k in range(F // 16):
                outb[e, pl.ds(k * 16, 16)] = zv
            return carry
        lax.fori_loop(0, C, zrow, 0, unroll=4)

    for p in range(P):
        zero_outb()
        for k in range(RPT // C):
            pltpu.sync_copy(outb, acc.at[pl.ds(row0 + k * C, C)])
        plsc.subcore_barrier()

        def load_pairs(c, B):
            e0 = wid * EPW + c * C
            return [
                (idxj_hbm.at[pl.ds(e0, C)], idxj[B]),
                (idxi_hbm.at[pl.ds(e0, C)], idxi[B]),
                (wf_hbm.at[pl.ds(p * E + e0, C)], wfb[B]),
                (uv1_hbm.at[pl.ds(3 * e0, 3 * C)], uv1b[B].at[pl.ds(0, 3 * C)]),
                (uv2_hbm.at[pl.ds(3 * e0, 3 * C)], uv2b[B].at[pl.ds(0, 3 * C)]),
            ]

        def issue_loads(c, B):
            for src, dst in load_pairs(c, B):
                pltpu.async_copy(src, dst, sem_l[B])

        def wait_loads(c, B):
            for src, dst in load_pairs(c, B):
                pltpu.make_async_copy(src, dst, sem_l[B]).wait()

        phi_p = phi_hbm.at[pl.ds(p * N, N)]
        v_p = v_hbm.at[pl.ds(p * N, N)]

        def issue_gathers(B):
            pltpu.async_copy(phi_p.at[idxj[B]], phib[B], sem_g[B])
            pltpu.async_copy(v_p.at[idxj[B]], vb[B], sem_g[B])

        def wait_gathers(B):
            pltpu.make_async_copy(
                phi_p.at[idxj[B]], phib[B], sem_g[B]).wait()
            pltpu.make_async_copy(
                v_p.at[idxj[B]], vb[B], sem_g[B]).wait()

        def compute(B):
            edge = make_edge(phib[B], wfb[B], vb[B], uv1b[B], uv2b[B])
            plsc.parallel_loop(0, C, unroll=2)(lambda e: edge(e, None))
            _scatter_add_rows(outb, acc, idxi[B])

        issue_loads(0, 0)
        wait_loads(0, 0)
        issue_gathers(0)
        issue_loads(1, 1)

        def pair(i, carry):
            c0 = 2 * i
            wait_loads(c0 + 1, 1)
            issue_gathers(1)
            wait_gathers(0)
            compute(0)
            issue_loads(c0 + 2, 0)
            wait_gathers(1)
            compute(1)
            issue_loads(c0 + 3, 1)
            wait_loads(c0 + 2, 0)
            issue_gathers(0)
            return carry

        lax.fori_loop(0, NCH // 2 - 1, pair, 0)
        wait_loads(NCH - 1, 1)
        issue_gathers(1)
        wait_gathers(0)
        compute(0)
        wait_gathers(1)
        compute(1)

        plsc.subcore_barrier()
        pltpu.sync_copy(
            acc.at[pl.ds(row0, RPT)],
            out_hbm.at[pl.ds((p * NCORES + cid) * NP + row0, RPT)])


def _make_sc_kernel():
    mesh = plsc.VectorSubcoreMesh(core_axis_name="c", subcore_axis_name="s",
                                  num_cores=NCORES, num_subcores=NSUB)
    out_type = jax.ShapeDtypeStruct((P * NCORES * NP, F), jnp.float32)
    scratch = (
        [pltpu.VMEM((C,), jnp.int32) for _ in range(2)]
        + [pltpu.VMEM((C,), jnp.int32) for _ in range(2)]
        + [pltpu.VMEM((C, GP), jnp.int32) for _ in range(2)]
        + [pltpu.VMEM((C, WP), jnp.int32) for _ in range(2)]
        + [pltpu.VMEM((C, VP), jnp.float32) for _ in range(2)]
        + [pltpu.VMEM((3 * C + 16,), jnp.float32) for _ in range(2)]
        + [pltpu.VMEM((3 * C + 16,), jnp.float32) for _ in range(2)]
        + [
            pltpu.VMEM((C, F), jnp.float32),
            pltpu.VMEM_SHARED((NP, F), jnp.float32),
            pltpu.SemaphoreType.DMA,
            pltpu.SemaphoreType.DMA,
            pltpu.SemaphoreType.DMA,
            pltpu.SemaphoreType.DMA,
        ])
    return pl.kernel(
        _sc_body, out_type=out_type, mesh=mesh, scratch_types=scratch,
        compiler_params=pltpu.CompilerParams(needs_layout_passes=False))


def _packed_perm():
    perm = (np.arange(6)[None, None, :, None] * F
            + np.arange(P)[:, None, None, None] * FQ
            + np.arange(2)[None, :, None, None] * 16
            + np.arange(16)[None, None, None, :])
    return perm.reshape(-1).astype(np.int32)


def kernel(s, v, radial_embeddings_1, radial_embeddings_2, f_cut_1, f_cut_2,
           unit_vectors_1, unit_vectors_2, edge_index, W1, b1, W2, b2, Wr,
           br):
    s2 = s.reshape(N, F)
    v2 = v.reshape(N, 3 * F)
    r1 = radial_embeddings_1.reshape(E, R)
    r2 = radial_embeddings_2.reshape(E, R)
    f1 = f_cut_1.reshape(E, 1)
    f2 = f_cut_2.reshape(E, 1)
    uv1 = unit_vectors_1.reshape(3 * E)
    uv2 = unit_vectors_2.reshape(3 * E)
    ei = edge_index.astype(jnp.int32)
    idx_i = ei[0]
    idx_j = ei[1]
    perm = _packed_perm()
    W2p = W2[:, perm]
    b2p = b2[perm].reshape(1, 6 * F)
    Wrp = Wr[:, perm]
    brp = br[perm].reshape(1, 6 * F)
    b1r = b1.reshape(1, F)

    phi4, v4 = pl.pallas_call(
        _phi_v_body,
        grid=(N // NB,),
        in_specs=[
            pl.BlockSpec((NB, F), lambda i: (i, 0)),
            pl.BlockSpec((NB, 3 * F), lambda i: (i, 0)),
            pl.BlockSpec((F, F), lambda i: (0, 0)),
            pl.BlockSpec((1, F), lambda i: (0, 0)),
            pl.BlockSpec((F, 6 * F), lambda i: (0, 0)),
            pl.BlockSpec((1, 6 * F), lambda i: (0, 0)),
        ],
        out_specs=[
            pl.BlockSpec((P, NB, GP), lambda i: (0, i, 0)),
            pl.BlockSpec((P, NB, VP), lambda i: (0, i, 0)),
        ],
        out_shape=[
            jax.ShapeDtypeStruct((P, N, GP), jnp.int32),
            jax.ShapeDtypeStruct((P, N, VP), jnp.float32),
        ],
    )(s2, v2, W1, b1r, W2p, b2p)

    wf4 = pl.pallas_call(
        _wfilt_body,
        grid=(E // EB,),
        in_specs=[
            pl.BlockSpec((EB, R), lambda i: (i, 0)),
            pl.BlockSpec((EB, R), lambda i: (i, 0)),
            pl.BlockSpec((EB, 1), lambda i: (i, 0)),
            pl.BlockSpec((EB, 1), lambda i: (i, 0)),
            pl.BlockSpec((R, 6 * F), lambda i: (0, 0)),
            pl.BlockSpec((1, 6 * F), lambda i: (0, 0)),
        ],
        out_specs=[pl.BlockSpec((P, EB, WP), lambda i: (0, i, 0))],
        out_shape=[jax.ShapeDtypeStruct((P, E, WP), jnp.int32)],
    )(r1, r2, f1, f2, Wrp, brp)
    wf4 = wf4[0]

    sc = _make_sc_kernel()
    parts = sc(phi4.reshape(P * N, GP), v4.reshape(P * N, VP),
               wf4.reshape(P * E, WP), idx_i, idx_j, uv1, uv2)
    parts = parts.reshape(P, NCORES, NP, F)

    out_s, out_v = pl.pallas_call(
        _combine_body,
        grid=(N // NB,),
        in_specs=[
            pl.BlockSpec((NB, F), lambda i: (i, 0)),
            pl.BlockSpec((NB, 3 * F), lambda i: (i, 0)),
            pl.BlockSpec((P, NCORES, NB, F), lambda i: (0, 0, i, 0)),
        ],
        out_specs=[
            pl.BlockSpec((NB, F), lambda i: (i, 0)),
            pl.BlockSpec((NB, 3 * F), lambda i: (i, 0)),
        ],
        out_shape=[
            jax.ShapeDtypeStruct((N, F), jnp.float32),
            jax.ShapeDtypeStruct((N, 3 * F), jnp.float32),
        ],
    )(s2, v2, parts)

    return (out_s.reshape(N, 1, F), out_v.reshape(N, 3, F))


# speedup vs baseline: 14.2091x; 1.0002x over previous
import jax
import jax.numpy as jnp
import numpy as np
from jax import lax
from jax.experimental import pallas as pl
from jax.experimental.pallas import tpu as pltpu
from jax.experimental.pallas import tpu_sc as plsc

N = 10000
E = 320000
F = 128
R = 32
P = 4
FQ = F // P
GW = 6 * FQ
VW = 3 * FQ
GP = 128
WP = 96
VP = 128
NB = 1000
EB = 2000
NCORES = 2
NSUB = 16
NWORK = NCORES * NSUB
EPW = E // NWORK
C = 40
NCH = EPW // C
NP = 10240
RPT = NP // NSUB

_MASK_HI = -65536


def _pack16(lo, hi):
    lob = lax.bitcast_convert_type(lo, jnp.int32)
    hib = lax.bitcast_convert_type(hi, jnp.int32)
    return jnp.bitwise_or(
        lax.shift_right_logical(lob + 32768, 16),
        jnp.bitwise_and(hib + 32768, _MASK_HI))


def _phi_v_body(s_ref, v_ref, w1_ref, b1_ref, w2_ref, b2_ref, phi_ref,
                v4_ref):
    h = jax.nn.silu(s_ref[...] @ w1_ref[...] + b1_ref[...])
    phi = h @ w2_ref[...] + b2_ref[...]
    vblk = v_ref[...]
    pad_g = jnp.zeros((phi.shape[0], GP - GW // 2), jnp.int32)
    pad_v = jnp.zeros((phi.shape[0], VP - VW), jnp.float32)
    for p in range(P):
        blk = phi[:, p * GW:(p + 1) * GW]
        phi_ref[p] = jnp.concatenate(
            [_pack16(blk[:, :GW // 2], blk[:, GW // 2:]), pad_g], axis=1)
        v4_ref[p] = jnp.concatenate(
            [vblk[:, d * F + p * FQ: d * F + (p + 1) * FQ] for d in range(3)]
            + [pad_v], axis=1)


def _wfilt_body(r1_ref, r2_ref, f1_ref, f2_ref, wr_ref, br_ref, wf_ref):
    f1 = f1_ref[...]
    f2 = f2_ref[...]
    rs = r1_ref[...] * f1 + r2_ref[...] * f2
    wf = rs @ wr_ref[...] + br_ref[...] * (f1 + f2)
    for p in range(P):
        blk = wf[:, p * GW:(p + 1) * GW]
        wf_ref[p] = _pack16(blk[:, :GW // 2], blk[:, GW // 2:])


def _combine_body(s_ref, v_ref, parts_ref, os_ref, ov_ref):
    ps = [parts_ref[p, 0] + parts_ref[p, 1] for p in range(P)]
    os_ref[...] = s_ref[...] + jnp.concatenate(
        [ps[p][:, 0:FQ] for p in range(P)], axis=1)
    ov_ref[...] = v_ref[...] + jnp.concatenate(
        [ps[p][:, FQ * (d + 1): FQ * (d + 2)] for d in range(3)
         for p in range(P)], axis=1)


def _scatter_add_rows(src, acc, idx):
    pltpu.sync_copy(src, acc.at[idx], add=True)


def _halves(buf, e, g):
    r = buf[e, pl.ds(g * 16, 16)]
    lo = plsc.bitcast(lax.shift_left(r, jnp.full((16,), 16, jnp.int32)),
                      jnp.float32)
    hi = plsc.bitcast(lax.bitwise_and(r, jnp.full((16,), _MASK_HI,
                                                  jnp.int32)),
                      jnp.float32)
    return lo, hi


def _sc_body(*refs):
    (phi_hbm, v_hbm, wf_hbm, idxi_hbm, idxj_hbm, uv1_hbm, uv2_hbm,
     out_hbm) = refs[0:8]
    sc = refs[8:]
    idxj = sc[0:2]
    idxi = sc[2:4]
    phib = sc[4:6]
    wfb = sc[6:8]
    vb = sc[8:10]
    uv1b = sc[10:12]
    uv2b = sc[12:14]
    outb = sc[14]
    acc = sc[15]
    sem_l = sc[16:18]
    sem_g = sc[18:20]

    cid = lax.axis_index("c")
    sid = lax.axis_index("s")
    wid = sid * NCORES + cid
    row0 = sid * RPT

    def make_edge(phib_, wfb_, vb_, uv1b_, uv2b_):
        def edge(e, carry):
            u1w = uv1b_[pl.ds(3 * e, 16)]
            u2w = uv2b_[pl.ds(3 * e, 16)]

            def bcast(win, d):
                dn = lax.GatherDimensionNumbers(
                    offset_dims=(), collapsed_slice_dims=(0,),
                    start_index_map=(0,))
                return lax.gather(
                    win, jnp.full((16, 1), d, jnp.int32), dn,
                    slice_sizes=(1,),
                    mode=lax.GatherScatterMode.PROMISE_IN_BOUNDS)
            u1 = [bcast(u1w, d) for d in range(3)]
            u2 = [bcast(u2w, d) for d in range(3)]
            x = {}
            for g in range(6):
                plo, phi_ = _halves(phib_, e, g)
                wlo, whi = _halves(wfb_, e, g)
                x[(g, 0)] = plo * wlo
                x[(g, 1)] = phi_ * whi
            for k in range(2):
                outb[e, pl.ds(k * 16, 16)] = x[(0, k)]
            vj = {}
            for d in range(3):
                for k in range(2):
                    vj[(d, k)] = vb_[e, pl.ds(d * FQ + k * 16, 16)]
            t = {}
            for d in range(3):
                for k in range(2):
                    t[(d, k)] = x[(4, k)] * u1[d] + x[(5, k)] * u2[d]
            for d in range(3):
                a = (d + 1) % 3
                b = (d + 2) % 3
                for k in range(2):
                    xv = (vj[(d, k)] * x[(1, k)] + x[(2, k)] * u1[d]
                          + x[(3, k)] * u2[d]
                          + vj[(a, k)] * t[(b, k)] - vj[(b, k)] * t[(a, k)])
                    outb[e, pl.ds(FQ + d * FQ + k * 16, 16)] = xv
            return carry
        return edge

    def zero_outb():
        zv = jnp.zeros((16,), jnp.float32)

        def zrow(e, carry):
            for k in range(F // 16):
                outb[e, pl.ds(k * 16, 16)] = zv
            return carry
        lax.fori_loop(0, C, zrow, 0, unroll=4)

    for p in range(P):
        zero_outb()
        for k in range(RPT // C):
            pltpu.sync_copy(outb, acc.at[pl.ds(row0 + k * C, C)])
        plsc.subcore_barrier()

        def load_pairs(c, B):
            e0 = wid * EPW + c * C
            return [
                (idxj_hbm.at[pl.ds(e0, C)], idxj[B]),
                (idxi_hbm.at[pl.ds(e0, C)], idxi[B]),
                (wf_hbm.at[pl.ds(p * E + e0, C)], wfb[B]),
                (uv1_hbm.at[pl.ds(3 * e0, 3 * C)], uv1b[B].at[pl.ds(0, 3 * C)]),
                (uv2_hbm.at[pl.ds(3 * e0, 3 * C)], uv2b[B].at[pl.ds(0, 3 * C)]),
            ]

        def issue_loads(c, B):
            for src, dst in load_pairs(c, B):
                pltpu.async_copy(src, dst, sem_l[B])

        def wait_loads(c, B):
            for src, dst in load_pairs(c, B):
                pltpu.make_async_copy(src, dst, sem_l[B]).wait()

        phi_p = phi_hbm.at[pl.ds(p * N, N)]
        v_p = v_hbm.at[pl.ds(p * N, N)]

        def issue_gathers(B):
            pltpu.async_copy(phi_p.at[idxj[B]], phib[B], sem_g[B])
            pltpu.async_copy(v_p.at[idxj[B]], vb[B], sem_g[B])

        def wait_gathers(B):
            pltpu.make_async_copy(
                phi_p.at[idxj[B]], phib[B], sem_g[B]).wait()
            pltpu.make_async_copy(
                v_p.at[idxj[B]], vb[B], sem_g[B]).wait()

        def compute(B):
            edge = make_edge(phib[B], wfb[B], vb[B], uv1b[B], uv2b[B])
            plsc.parallel_loop(0, C, unroll=2)(lambda e: edge(e, None))
            _scatter_add_rows(outb, acc, idxi[B])

        issue_loads(0, 0)
        wait_loads(0, 0)
        issue_gathers(0)
        issue_loads(1, 1)

        def pair(i, carry):
            c0 = 2 * i
            wait_loads(c0 + 1, 1)
            issue_gathers(1)
            wait_gathers(0)
            compute(0)
            issue_loads(c0 + 2, 0)
            wait_gathers(1)
            compute(1)
            issue_loads(c0 + 3, 1)
            wait_loads(c0 + 2, 0)
            issue_gathers(0)
            return carry

        lax.fori_loop(0, NCH // 2 - 1, pair, 0)
        wait_loads(NCH - 1, 1)
        issue_gathers(1)
        wait_gathers(0)
        compute(0)
        wait_gathers(1)
        compute(1)

        plsc.subcore_barrier()
        pltpu.sync_copy(
            acc.at[pl.ds(row0, RPT)],
            out_hbm.at[pl.ds((p * NCORES + cid) * NP + row0, RPT)])


def _make_sc_kernel():
    mesh = plsc.VectorSubcoreMesh(core_axis_name="c", subcore_axis_name="s",
                                  num_cores=NCORES, num_subcores=NSUB)
    out_type = jax.ShapeDtypeStruct((P * NCORES * NP, F), jnp.float32)
    scratch = (
        [pltpu.VMEM((C,), jnp.int32) for _ in range(2)]
        + [pltpu.VMEM((C,), jnp.int32) for _ in range(2)]
        + [pltpu.VMEM((C, GP), jnp.int32) for _ in range(2)]
        + [pltpu.VMEM((C, WP), jnp.int32) for _ in range(2)]
        + [pltpu.VMEM((C, VP), jnp.float32) for _ in range(2)]
        + [pltpu.VMEM((3 * C + 16,), jnp.float32) for _ in range(2)]
        + [pltpu.VMEM((3 * C + 16,), jnp.float32) for _ in range(2)]
        + [
            pltpu.VMEM((C, F), jnp.float32),
            pltpu.VMEM_SHARED((NP, F), jnp.float32),
            pltpu.SemaphoreType.DMA,
            pltpu.SemaphoreType.DMA,
            pltpu.SemaphoreType.DMA,
            pltpu.SemaphoreType.DMA,
        ])
    return pl.kernel(
        _sc_body, out_type=out_type, mesh=mesh, scratch_types=scratch,
        compiler_params=pltpu.CompilerParams(needs_layout_passes=False))


def _packed_perm():
    perm = (np.arange(6)[None, None, :, None] * F
            + np.arange(P)[:, None, None, None] * FQ
            + np.arange(2)[None, :, None, None] * 16
            + np.arange(16)[None, None, None, :])
    return perm.reshape(-1).astype(np.int32)


def kernel(s, v, radial_embeddings_1, radial_embeddings_2, f_cut_1, f_cut_2,
           unit_vectors_1, unit_vectors_2, edge_index, W1, b1, W2, b2, Wr,
           br):
    s2 = s.reshape(N, F)
    v2 = v.reshape(N, 3 * F)
    r1 = radial_embeddings_1.reshape(E, R)
    r2 = radial_embeddings_2.reshape(E, R)
    f1 = f_cut_1.reshape(E, 1)
    f2 = f_cut_2.reshape(E, 1)
    uv1 = unit_vectors_1.reshape(3 * E)
    uv2 = unit_vectors_2.reshape(3 * E)
    ei = edge_index.astype(jnp.int32)
    idx_i = ei[0]
    idx_j = ei[1]
    perm = _packed_perm()
    W2p = W2[:, perm]
    b2p = b2[perm].reshape(1, 6 * F)
    Wrp = Wr[:, perm]
    brp = br[perm].reshape(1, 6 * F)
    b1r = b1.reshape(1, F)

    phi4, v4 = pl.pallas_call(
        _phi_v_body,
        grid=(N // NB,),
        in_specs=[
            pl.BlockSpec((NB, F), lambda i: (i, 0)),
            pl.BlockSpec((NB, 3 * F), lambda i: (i, 0)),
            pl.BlockSpec((F, F), lambda i: (0, 0)),
            pl.BlockSpec((1, F), lambda i: (0, 0)),
            pl.BlockSpec((F, 6 * F), lambda i: (0, 0)),
            pl.BlockSpec((1, 6 * F), lambda i: (0, 0)),
        ],
        out_specs=[
            pl.BlockSpec((P, NB, GP), lambda i: (0, i, 0)),
            pl.BlockSpec((P, NB, VP), lambda i: (0, i, 0)),
        ],
        out_shape=[
            jax.ShapeDtypeStruct((P, N, GP), jnp.int32),
            jax.ShapeDtypeStruct((P, N, VP), jnp.float32),
        ],
    )(s2, v2, W1, b1r, W2p, b2p)

    wf4 = pl.pallas_call(
        _wfilt_body,
        grid=(E // EB,),
        in_specs=[
            pl.BlockSpec((EB, R), lambda i: (i, 0)),
            pl.BlockSpec((EB, R), lambda i: (i, 0)),
            pl.BlockSpec((EB, 1), lambda i: (i, 0)),
            pl.BlockSpec((EB, 1), lambda i: (i, 0)),
            pl.BlockSpec((R, 6 * F), lambda i: (0, 0)),
            pl.BlockSpec((1, 6 * F), lambda i: (0, 0)),
        ],
        out_specs=[pl.BlockSpec((P, EB, WP), lambda i: (0, i, 0))],
        out_shape=[jax.ShapeDtypeStruct((P, E, WP), jnp.int32)],
    )(r1, r2, f1, f2, Wrp, brp)
    wf4 = wf4[0]

    sc = _make_sc_kernel()
    parts = sc(phi4.reshape(P * N, GP), v4.reshape(P * N, VP),
               wf4.reshape(P * E, WP), idx_i, idx_j, uv1, uv2)
    parts = parts.reshape(P, NCORES, NP, F)

    out_s, out_v = pl.pallas_call(
        _combine_body,
        grid=(N // NB,),
        in_specs=[
            pl.BlockSpec((NB, F), lambda i: (i, 0)),
            pl.BlockSpec((NB, 3 * F), lambda i: (i, 0)),
            pl.BlockSpec((P, NCORES, NB, F), lambda i: (0, 0, i, 0)),
        ],
        out_specs=[
            pl.BlockSpec((NB, F), lambda i: (i, 0)),
            pl.BlockSpec((NB, 3 * F), lambda i: (i, 0)),
        ],
        out_shape=[
            jax.ShapeDtypeStruct((N, F), jnp.float32),
            jax.ShapeDtypeStruct((N, 3 * F), jnp.float32),
        ],
    )(s2, v2, parts)

    return (out_s.reshape(N, 1, F), out_v.reshape(N, 3, F))
